# Initial kernel scaffold; baseline (speedup 1.0000x reference)
import jax
import jax.numpy as jnp
from jax.experimental import pallas as pl


def kernel(x, edge_index, batch, W1, b1, W2, b2, fcW1, fcb1, fcW2, fcb2):
    raise NotImplementedError("write your pallas kernel here")


# speedup vs baseline: 12.6825x; 12.6825x over previous
import functools

import jax
import jax.numpy as jnp
from jax import lax
from jax.experimental import pallas as pl
from jax.experimental.pallas import tpu as pltpu
from jax.experimental.pallas import tpu_sc as plsc

N = 10000
E = 320000
IN = 128
H1 = 64
H2 = 128
FC1 = 1024
OUT = 6400
G = 64

NC, NS = 2, 16
NW = NC * NS
NPAD = 10240
RPT = NPAD // NS
EW = E // NW
CH = 80
NCHUNK = EW // CH

BLK = 512
NB = NPAD // BLK

def _deg_body(dst_hbm, zd_hbm, dp_hbm, didx, ones_v, acc):
    c = lax.axis_index("c")
    s = lax.axis_index("s")
    wid = s * NC + c
    pltpu.sync_copy(zd_hbm.at[pl.ds(s * RPT, RPT)], acc.at[pl.ds(s * RPT, RPT)])
    for i in range(CH // 16):
        ones_v[pl.ds(i * 16, 16)] = jnp.full((16,), 1.0, jnp.float32)
    plsc.subcore_barrier()
    base = wid * EW

    def step(j, carry):
        off = base + j * CH
        pltpu.sync_copy(dst_hbm.at[pl.ds(off, CH)], didx)
        pltpu.sync_copy(ones_v, acc.at[didx], add=True)
        return carry

    lax.fori_loop(0, NCHUNK, step, 0)
    plsc.subcore_barrier()
    pltpu.sync_copy(acc.at[pl.ds(s * RPT, RPT)], dp_hbm.at[c, pl.ds(s * RPT, RPT)])


def _scat_body(hs_hbm, src_hbm, dst_hbm, zr_hbm, part_hbm, sidx, didx, rows, acc, sem):
    c = lax.axis_index("c")
    s = lax.axis_index("s")
    wid = s * NC + c
    pltpu.sync_copy(zr_hbm.at[pl.ds(s * RPT, RPT)], acc.at[pl.ds(s * RPT, RPT)])
    plsc.subcore_barrier()
    base = wid * EW

    def step(j, carry):
        off = base + j * CH
        pltpu.sync_copy(src_hbm.at[pl.ds(off, CH)], sidx)
        pltpu.sync_copy(dst_hbm.at[pl.ds(off, CH)], didx)
        pltpu.async_copy(hs_hbm.at[sidx], rows, sem).wait()
        pltpu.sync_copy(rows, acc.at[didx], add=True)
        return carry

    lax.fori_loop(0, NCHUNK, step, 0)
    plsc.subcore_barrier()
    pltpu.sync_copy(acc.at[pl.ds(s * RPT, RPT)],
                    part_hbm.at[c, pl.ds(s * RPT, RPT)])


@functools.lru_cache(maxsize=None)
def _sc_kernels():
    mesh = plsc.VectorSubcoreMesh(core_axis_name="c", subcore_axis_name="s",
                                  num_cores=NC, num_subcores=NS)
    deg = pl.kernel(
        _deg_body,
        out_type=jax.ShapeDtypeStruct((NC, NPAD), jnp.float32),
        mesh=mesh,
        scratch_types=[
            pltpu.VMEM((CH,), jnp.int32),
            pltpu.VMEM((CH,), jnp.float32),
            pltpu.VMEM_SHARED((NPAD,), jnp.float32),
        ],
    )

    scat = pl.kernel(
        _scat_body,
        out_type=jax.ShapeDtypeStruct((NC, NPAD, H2), jnp.float32),
        mesh=mesh,
        scratch_types=[
            pltpu.VMEM((CH,), jnp.int32),
            pltpu.VMEM((CH,), jnp.int32),
            pltpu.VMEM((CH, H2), jnp.float32),
            pltpu.VMEM_SHARED((NPAD, H2), jnp.float32),
            pltpu.SemaphoreType.DMA,
        ],
    )

    return deg, scat


def _dinv_of(dp_ref):
    deg = dp_ref[0, :] + dp_ref[1, :] + 1.0
    return lax.rsqrt(deg)[:, None]


def _tc1_body(x_ref, w1_ref, dp_ref, hs1_ref):
    dinv = _dinv_of(dp_ref)
    hs1 = dinv * jnp.dot(x_ref[...], w1_ref[...],
                         preferred_element_type=jnp.float32)
    hs1_ref[...] = jnp.concatenate(
        [hs1, jnp.zeros((BLK, H2 - H1), jnp.float32)], axis=1)


def _tc2_body(p1_ref, hs1_ref, dp_ref, b1_ref, w2_ref, hs2_ref):
    dinv = _dinv_of(dp_ref)
    agg = (p1_ref[0] + p1_ref[1] + hs1_ref[...])[:, :H1]
    h = jnp.maximum(dinv * agg + b1_ref[...], 0.0)
    hs2_ref[...] = dinv * jnp.dot(h, w2_ref[...],
                                  preferred_element_type=jnp.float32)


def _tc3_body(p2_ref, hs2_ref, dp_ref, b2_ref, bat_ref, psum_ref, cnt_ref):
    i = pl.program_id(0)
    dinv = _dinv_of(dp_ref)
    h2 = jnp.maximum(dinv * (p2_ref[0] + p2_ref[1] + hs2_ref[...]) + b2_ref[...],
                     0.0)
    gid = lax.broadcasted_iota(jnp.int32, (BLK, G), 1)
    oh = (bat_ref[...] == gid).astype(jnp.float32)

    @pl.when(i == 0)
    def _():
        psum_ref[...] = jnp.zeros_like(psum_ref)
        cnt_ref[...] = jnp.zeros_like(cnt_ref)

    psum_ref[...] += lax.dot_general(oh, h2, (((0,), (0,)), ((), ())),
                                     preferred_element_type=jnp.float32)
    cnt_ref[...] += lax.dot_general(oh, jnp.ones((BLK, H2), jnp.float32),
                                    (((0,), (0,)), ((), ())),
                                    preferred_element_type=jnp.float32)


def _tc4_body(psum_ref, cnt_ref, fw1_ref, fb1_ref, fw2_ref, fb2_ref, out_ref):
    pooled = psum_ref[...] / jnp.maximum(cnt_ref[...], 1.0)
    z = jnp.maximum(jnp.dot(pooled, fw1_ref[...],
                            preferred_element_type=jnp.float32) + fb1_ref[...],
                    0.0)
    out_ref[...] = jnp.dot(z, fw2_ref[...],
                           preferred_element_type=jnp.float32) + fb2_ref[...]


_tc1 = pl.pallas_call(
    _tc1_body,
    grid=(NB,),
    in_specs=[
        pl.BlockSpec((BLK, IN), lambda i: (i, 0)),
        pl.BlockSpec((IN, H1), lambda i: (0, 0)),
        pl.BlockSpec((2, BLK), lambda i: (0, i)),
    ],
    out_specs=pl.BlockSpec((BLK, H2), lambda i: (i, 0)),
    out_shape=jax.ShapeDtypeStruct((NPAD, H2), jnp.float32),
)

_tc2 = pl.pallas_call(
    _tc2_body,
    grid=(NB,),
    in_specs=[
        pl.BlockSpec((2, BLK, H2), lambda i: (0, i, 0)),
        pl.BlockSpec((BLK, H2), lambda i: (i, 0)),
        pl.BlockSpec((2, BLK), lambda i: (0, i)),
        pl.BlockSpec((1, H1), lambda i: (0, 0)),
        pl.BlockSpec((H1, H2), lambda i: (0, 0)),
    ],
    out_specs=pl.BlockSpec((BLK, H2), lambda i: (i, 0)),
    out_shape=jax.ShapeDtypeStruct((NPAD, H2), jnp.float32),
)

_tc3 = pl.pallas_call(
    _tc3_body,
    grid=(NB,),
    in_specs=[
        pl.BlockSpec((2, BLK, H2), lambda i: (0, i, 0)),
        pl.BlockSpec((BLK, H2), lambda i: (i, 0)),
        pl.BlockSpec((2, BLK), lambda i: (0, i)),
        pl.BlockSpec((1, H2), lambda i: (0, 0)),
        pl.BlockSpec((BLK, 1), lambda i: (i, 0)),
    ],
    out_specs=[
        pl.BlockSpec((G, H2), lambda i: (0, 0)),
        pl.BlockSpec((G, H2), lambda i: (0, 0)),
    ],
    out_shape=[
        jax.ShapeDtypeStruct((G, H2), jnp.float32),
        jax.ShapeDtypeStruct((G, H2), jnp.float32),
    ],
)

_OB = OUT // 10
_tc4 = pl.pallas_call(
    _tc4_body,
    grid=(10,),
    in_specs=[
        pl.BlockSpec((G, H2), lambda i: (0, 0)),
        pl.BlockSpec((G, H2), lambda i: (0, 0)),
        pl.BlockSpec((H2, FC1), lambda i: (0, 0)),
        pl.BlockSpec((1, FC1), lambda i: (0, 0)),
        pl.BlockSpec((FC1, _OB), lambda i: (0, i)),
        pl.BlockSpec((1, _OB), lambda i: (0, i)),
    ],
    out_specs=pl.BlockSpec((G, _OB), lambda i: (0, i)),
    out_shape=jax.ShapeDtypeStruct((G, OUT), jnp.float32),
)


def kernel(x, edge_index, batch, W1, b1, W2, b2, fcW1, fcb1, fcW2, fcb2):
    src = edge_index[0]
    dst = edge_index[1]
    x_p = jnp.pad(x, ((0, NPAD - N), (0, 0)))
    bat_p = jnp.pad(batch, (0, NPAD - N), constant_values=G).reshape(NPAD, 1)
    zd = jnp.zeros((NPAD,), jnp.float32)
    z2 = jnp.zeros((NPAD, H2), jnp.float32)

    _deg, _scat = _sc_kernels()
    dp = _deg(dst, zd)
    hs1 = _tc1(x_p, W1, dp)
    p1 = _scat(hs1, src, dst, z2)
    hs2 = _tc2(p1, hs1, dp, b1.reshape(1, H1), W2)
    p2 = _scat(hs2, src, dst, z2)
    psum, cnt = _tc3(p2, hs2, dp, b2.reshape(1, H2), bat_p)
    out = _tc4(psum, cnt, fcW1, fcb1.reshape(1, FC1), fcW2, fcb2.reshape(1, OUT))
    return out


# speedup vs baseline: 14.8212x; 1.1686x over previous
import functools

import jax
import jax.numpy as jnp
from jax import lax
from jax.experimental import pallas as pl
from jax.experimental.pallas import tpu as pltpu
from jax.experimental.pallas import tpu_sc as plsc

N = 10000
E = 320000
IN = 128
H1 = 64
H2 = 128
FC1 = 1024
OUT = 6400
G = 64

NC, NS = 2, 16
NW = NC * NS
NPAD = 10240
RPT = NPAD // NS
EW = E // NW
CH = 80
NCHUNK = EW // CH

BLK = 512
NB = NPAD // BLK

def _deg_body(dst_hbm, zd_hbm, dp_hbm, didx0, didx1, ones_v, acc, ssem0, ssem1):
    c = lax.axis_index("c")
    s = lax.axis_index("s")
    wid = s * NC + c
    pltpu.sync_copy(zd_hbm.at[pl.ds(s * RPT, RPT)], acc.at[pl.ds(s * RPT, RPT)])
    for i in range(CH // 16):
        ones_v[pl.ds(i * 16, 16)] = jnp.full((16,), 1.0, jnp.float32)
    plsc.subcore_barrier()
    base = wid * EW
    bufs = (didx0, didx1)
    sems = (ssem0, ssem1)
    pltpu.sync_copy(dst_hbm.at[pl.ds(base, CH)], didx0)

    def substep(t, b):
        cur, oth = bufs[b], bufs[1 - b]
        pltpu.async_copy(ones_v, acc.at[cur], sems[b], add=True)

        @pl.when(t > 0)
        def _():
            pltpu.make_async_copy(ones_v, acc.at[oth], sems[1 - b]).wait()

        @pl.when(t < NCHUNK - 1)
        def _():
            pltpu.sync_copy(dst_hbm.at[pl.ds(base + (t + 1) * CH, CH)], oth)

    def step(jj, carry):
        substep(2 * jj, 0)
        substep(2 * jj + 1, 1)
        return carry

    lax.fori_loop(0, NCHUNK // 2, step, 0)
    if NCHUNK % 2:
        substep(NCHUNK - 1, 0)
    lb = (NCHUNK - 1) % 2
    pltpu.make_async_copy(ones_v, acc.at[bufs[lb]], sems[lb]).wait()
    plsc.subcore_barrier()
    pltpu.sync_copy(acc.at[pl.ds(s * RPT, RPT)], dp_hbm.at[c, pl.ds(s * RPT, RPT)])


def _scat_body(hs_hbm, src_hbm, dst_hbm, zr_hbm, part_hbm,
               sidx0, didx0, sidx1, didx1, rows0, rows1, acc, gsem, ssem0, ssem1):
    c = lax.axis_index("c")
    s = lax.axis_index("s")
    wid = s * NC + c
    pltpu.sync_copy(zr_hbm.at[pl.ds(s * RPT, RPT)], acc.at[pl.ds(s * RPT, RPT)])
    plsc.subcore_barrier()
    base = wid * EW
    sbufs = (sidx0, sidx1)
    dbufs = (didx0, didx1)
    rbufs = (rows0, rows1)
    sems = (ssem0, ssem1)

    pltpu.sync_copy(src_hbm.at[pl.ds(base, CH)], sidx0)
    pltpu.sync_copy(dst_hbm.at[pl.ds(base, CH)], didx0)
    pltpu.async_copy(hs_hbm.at[sidx0], rows0, gsem)

    def substep(t, b):
        sc, dc, rc = sbufs[b], dbufs[b], rbufs[b]
        so, do, ro = sbufs[1 - b], dbufs[1 - b], rbufs[1 - b]
        pltpu.make_async_copy(hs_hbm.at[sc], rc, gsem).wait()
        pltpu.async_copy(rc, acc.at[dc], sems[b], add=True)

        @pl.when(t > 0)
        def _():
            pltpu.make_async_copy(ro, acc.at[do], sems[1 - b]).wait()

        @pl.when(t < NCHUNK - 1)
        def _():
            off = base + (t + 1) * CH
            pltpu.sync_copy(src_hbm.at[pl.ds(off, CH)], so)
            pltpu.sync_copy(dst_hbm.at[pl.ds(off, CH)], do)
            pltpu.async_copy(hs_hbm.at[so], ro, gsem)

    def step(jj, carry):
        substep(2 * jj, 0)
        substep(2 * jj + 1, 1)
        return carry

    lax.fori_loop(0, NCHUNK // 2, step, 0)
    if NCHUNK % 2:
        substep(NCHUNK - 1, 0)
    lb = (NCHUNK - 1) % 2
    pltpu.make_async_copy(rbufs[lb], acc.at[dbufs[lb]], sems[lb]).wait()
    plsc.subcore_barrier()
    pltpu.sync_copy(acc.at[pl.ds(s * RPT, RPT)],
                    part_hbm.at[c, pl.ds(s * RPT, RPT)])


@functools.lru_cache(maxsize=None)
def _sc_kernels():
    mesh = plsc.VectorSubcoreMesh(core_axis_name="c", subcore_axis_name="s",
                                  num_cores=NC, num_subcores=NS)
    deg = pl.kernel(
        _deg_body,
        out_type=jax.ShapeDtypeStruct((NC, NPAD), jnp.float32),
        mesh=mesh,
        scratch_types=[
            pltpu.VMEM((CH,), jnp.int32),
            pltpu.VMEM((CH,), jnp.int32),
            pltpu.VMEM((CH,), jnp.float32),
            pltpu.VMEM_SHARED((NPAD,), jnp.float32),
            pltpu.SemaphoreType.DMA,
            pltpu.SemaphoreType.DMA,
        ],
    )

    scat = pl.kernel(
        _scat_body,
        out_type=jax.ShapeDtypeStruct((NC, NPAD, H2), jnp.float32),
        mesh=mesh,
        scratch_types=[
            pltpu.VMEM((CH,), jnp.int32),
            pltpu.VMEM((CH,), jnp.int32),
            pltpu.VMEM((CH,), jnp.int32),
            pltpu.VMEM((CH,), jnp.int32),
            pltpu.VMEM((CH, H2), jnp.float32),
            pltpu.VMEM((CH, H2), jnp.float32),
            pltpu.VMEM_SHARED((NPAD, H2), jnp.float32),
            pltpu.SemaphoreType.DMA,
            pltpu.SemaphoreType.DMA,
            pltpu.SemaphoreType.DMA,
        ],
    )

    return deg, scat


def _dinv_of(dp_ref):
    deg = dp_ref[0, :] + dp_ref[1, :] + 1.0
    return lax.rsqrt(deg)[:, None]


def _tc1_body(x_ref, w1_ref, dp_ref, hs1_ref):
    dinv = _dinv_of(dp_ref)
    hs1 = dinv * jnp.dot(x_ref[...], w1_ref[...],
                         preferred_element_type=jnp.float32)
    hs1_ref[...] = jnp.concatenate(
        [hs1, jnp.zeros((BLK, H2 - H1), jnp.float32)], axis=1)


def _tc2_body(p1_ref, hs1_ref, dp_ref, b1_ref, w2_ref, hs2_ref):
    dinv = _dinv_of(dp_ref)
    agg = (p1_ref[0] + p1_ref[1] + hs1_ref[...])[:, :H1]
    h = jnp.maximum(dinv * agg + b1_ref[...], 0.0)
    hs2_ref[...] = dinv * jnp.dot(h, w2_ref[...],
                                  preferred_element_type=jnp.float32)


def _tc3_body(p2_ref, hs2_ref, dp_ref, b2_ref, bat_ref, psum_ref, cnt_ref):
    i = pl.program_id(0)
    dinv = _dinv_of(dp_ref)
    h2 = jnp.maximum(dinv * (p2_ref[0] + p2_ref[1] + hs2_ref[...]) + b2_ref[...],
                     0.0)
    gid = lax.broadcasted_iota(jnp.int32, (BLK, G), 1)
    oh = (bat_ref[...] == gid).astype(jnp.float32)

    @pl.when(i == 0)
    def _():
        psum_ref[...] = jnp.zeros_like(psum_ref)
        cnt_ref[...] = jnp.zeros_like(cnt_ref)

    psum_ref[...] += lax.dot_general(oh, h2, (((0,), (0,)), ((), ())),
                                     preferred_element_type=jnp.float32)
    cnt_ref[...] += lax.dot_general(oh, jnp.ones((BLK, H2), jnp.float32),
                                    (((0,), (0,)), ((), ())),
                                    preferred_element_type=jnp.float32)


def _tc4_body(psum_ref, cnt_ref, fw1_ref, fb1_ref, fw2_ref, fb2_ref, out_ref):
    pooled = psum_ref[...] / jnp.maximum(cnt_ref[...], 1.0)
    z = jnp.maximum(jnp.dot(pooled, fw1_ref[...],
                            preferred_element_type=jnp.float32) + fb1_ref[...],
                    0.0)
    out_ref[...] = jnp.dot(z, fw2_ref[...],
                           preferred_element_type=jnp.float32) + fb2_ref[...]


_tc1 = pl.pallas_call(
    _tc1_body,
    grid=(NB,),
    in_specs=[
        pl.BlockSpec((BLK, IN), lambda i: (i, 0)),
        pl.BlockSpec((IN, H1), lambda i: (0, 0)),
        pl.BlockSpec((2, BLK), lambda i: (0, i)),
    ],
    out_specs=pl.BlockSpec((BLK, H2), lambda i: (i, 0)),
    out_shape=jax.ShapeDtypeStruct((NPAD, H2), jnp.float32),
)

_tc2 = pl.pallas_call(
    _tc2_body,
    grid=(NB,),
    in_specs=[
        pl.BlockSpec((2, BLK, H2), lambda i: (0, i, 0)),
        pl.BlockSpec((BLK, H2), lambda i: (i, 0)),
        pl.BlockSpec((2, BLK), lambda i: (0, i)),
        pl.BlockSpec((1, H1), lambda i: (0, 0)),
        pl.BlockSpec((H1, H2), lambda i: (0, 0)),
    ],
    out_specs=pl.BlockSpec((BLK, H2), lambda i: (i, 0)),
    out_shape=jax.ShapeDtypeStruct((NPAD, H2), jnp.float32),
)

_tc3 = pl.pallas_call(
    _tc3_body,
    grid=(NB,),
    in_specs=[
        pl.BlockSpec((2, BLK, H2), lambda i: (0, i, 0)),
        pl.BlockSpec((BLK, H2), lambda i: (i, 0)),
        pl.BlockSpec((2, BLK), lambda i: (0, i)),
        pl.BlockSpec((1, H2), lambda i: (0, 0)),
        pl.BlockSpec((BLK, 1), lambda i: (i, 0)),
    ],
    out_specs=[
        pl.BlockSpec((G, H2), lambda i: (0, 0)),
        pl.BlockSpec((G, H2), lambda i: (0, 0)),
    ],
    out_shape=[
        jax.ShapeDtypeStruct((G, H2), jnp.float32),
        jax.ShapeDtypeStruct((G, H2), jnp.float32),
    ],
)

_OB = OUT // 10
_tc4 = pl.pallas_call(
    _tc4_body,
    grid=(10,),
    in_specs=[
        pl.BlockSpec((G, H2), lambda i: (0, 0)),
        pl.BlockSpec((G, H2), lambda i: (0, 0)),
        pl.BlockSpec((H2, FC1), lambda i: (0, 0)),
        pl.BlockSpec((1, FC1), lambda i: (0, 0)),
        pl.BlockSpec((FC1, _OB), lambda i: (0, i)),
        pl.BlockSpec((1, _OB), lambda i: (0, i)),
    ],
    out_specs=pl.BlockSpec((G, _OB), lambda i: (0, i)),
    out_shape=jax.ShapeDtypeStruct((G, OUT), jnp.float32),
)


def kernel(x, edge_index, batch, W1, b1, W2, b2, fcW1, fcb1, fcW2, fcb2):
    x_p = jnp.pad(x, ((0, NPAD - N), (0, 0)))
    bat_p = jnp.pad(batch, (0, NPAD - N), constant_values=G).reshape(NPAD, 1)
    zd = jnp.zeros((NPAD,), jnp.float32)
    z2 = jnp.zeros((NPAD, H2), jnp.float32)

    src = edge_index[0]
    dst = edge_index[1]
    _deg, _scat = _sc_kernels()
    dp = _deg(dst, zd)
    hs1 = _tc1(x_p, W1, dp)
    p1 = _scat(hs1, src, dst, z2)
    hs2 = _tc2(p1, hs1, dp, b1.reshape(1, H1), W2)
    p2 = _scat(hs2, src, dst, z2)
    psum, cnt = _tc3(p2, hs2, dp, b2.reshape(1, H2), bat_p)
    out = _tc4(psum, cnt, fcW1, fcb1.reshape(1, FC1), fcW2, fcb2.reshape(1, OUT))
    return out


# speedup vs baseline: 21.2571x; 1.4342x over previous
import functools

import jax
import jax.numpy as jnp
from jax import lax
from jax.experimental import pallas as pl
from jax.experimental.pallas import tpu as pltpu
from jax.experimental.pallas import tpu_sc as plsc

N = 10000
E = 320000
IN = 128
H1 = 64
H2 = 128
FC1 = 1024
OUT = 6400
G = 64

NC, NS = 2, 16
NW = NC * NS
NPAD = 10240
RPT = NPAD // NS
EW = E // NW
CH = 80
NCHUNK = EW // CH

BLK = 512
NB = NPAD // BLK

def _deg_body(dst_hbm, zd_hbm, dp_hbm, didx0, didx1, ones_v, acc,
              ssem0, ssem1, isem0, isem1):
    c = lax.axis_index("c")
    s = lax.axis_index("s")
    wid = s * NC + c
    pltpu.sync_copy(zd_hbm.at[pl.ds(s * RPT, RPT)], acc.at[pl.ds(s * RPT, RPT)])
    for i in range(CH // 16):
        ones_v[pl.ds(i * 16, 16)] = jnp.full((16,), 1.0, jnp.float32)
    plsc.subcore_barrier()
    base = wid * EW
    bufs = (didx0, didx1)
    sems = (ssem0, ssem1)
    isems = (isem0, isem1)

    def idx_issue(t, k):
        pltpu.async_copy(dst_hbm.at[pl.ds(base + t * CH, CH)], bufs[k], isems[k])

    def idx_wait(t, k):
        pltpu.make_async_copy(dst_hbm.at[pl.ds(base + t * CH, CH)],
                              bufs[k], isems[k]).wait()

    def substep(t, k, first=False):
        idx_wait(t, k)
        pltpu.async_copy(ones_v, acc.at[bufs[k]], sems[k], add=True)
        if not first:
            pltpu.make_async_copy(ones_v, acc.at[bufs[1 - k]], sems[1 - k]).wait()

        @pl.when(t < NCHUNK - 1)
        def _():
            idx_issue(t + 1, 1 - k)

    idx_issue(0, 0)
    substep(0, 0, first=True)

    def step(jj, carry):
        substep(2 * jj + 1, 1)
        substep(2 * jj + 2, 0)
        return carry

    lax.fori_loop(0, (NCHUNK - 1) // 2, step, 0)
    pltpu.make_async_copy(ones_v, acc.at[bufs[0]], sems[0]).wait()
    plsc.subcore_barrier()
    pltpu.sync_copy(acc.at[pl.ds(s * RPT, RPT)], dp_hbm.at[c, pl.ds(s * RPT, RPT)])


def _scat_body(hs_hbm, src_hbm, dst_hbm, zr_hbm, part_hbm,
               si0, si1, si2, di0, di1, di2, rows0, rows1, acc,
               gsem, ssem0, ssem1, isem0, isem1, isem2):
    c = lax.axis_index("c")
    s = lax.axis_index("s")
    wid = s * NC + c
    pltpu.sync_copy(zr_hbm.at[pl.ds(s * RPT, RPT)], acc.at[pl.ds(s * RPT, RPT)])
    plsc.subcore_barrier()
    base = wid * EW
    sib = (si0, si1, si2)
    dib = (di0, di1, di2)
    rbufs = (rows0, rows1)
    ssems = (ssem0, ssem1)
    isems = (isem0, isem1, isem2)

    def idx_issue(t, k3):
        off = base + t * CH
        pltpu.async_copy(src_hbm.at[pl.ds(off, CH)], sib[k3], isems[k3])
        pltpu.async_copy(dst_hbm.at[pl.ds(off, CH)], dib[k3], isems[k3])

    def idx_wait(t, k3):
        off = base + t * CH
        pltpu.make_async_copy(src_hbm.at[pl.ds(off, CH)], sib[k3], isems[k3]).wait()
        pltpu.make_async_copy(dst_hbm.at[pl.ds(off, CH)], dib[k3], isems[k3]).wait()

    def substep(t, k2, k3, first=False, gather_next=True, idx_next=True):
        rc, ro = rbufs[k2], rbufs[1 - k2]
        pltpu.make_async_copy(hs_hbm.at[sib[k3]], rc, gsem).wait()
        pltpu.async_copy(rc, acc.at[dib[k3]], ssems[k2], add=True)
        if not first:
            pltpu.make_async_copy(ro, acc.at[dib[(k3 + 2) % 3]],
                                  ssems[1 - k2]).wait()
        if gather_next:
            idx_wait(t + 1, (k3 + 1) % 3)
            pltpu.async_copy(hs_hbm.at[sib[(k3 + 1) % 3]], ro, gsem)
        if idx_next:
            idx_issue(t + 2, (k3 + 2) % 3)

    idx_issue(0, 0)
    idx_issue(1, 1)
    idx_wait(0, 0)
    pltpu.async_copy(hs_hbm.at[si0], rows0, gsem)
    substep(0, 0, 0, first=True)
    for t in range(1, 6):
        substep(t, t % 2, t % 3)

    def step(jj, carry):
        t = 6 * jj
        for k in range(6):
            substep(t + k, k % 2, k % 3)
        return carry

    lax.fori_loop(1, (NCHUNK - 5) // 6, step, 0)
    for t in range(NCHUNK - 5, NCHUNK):
        substep(t, t % 2, t % 3,
                gather_next=(t + 1 < NCHUNK), idx_next=(t + 2 < NCHUNK))
    lb = (NCHUNK - 1) % 2
    pltpu.make_async_copy(rbufs[lb], acc.at[dib[(NCHUNK - 1) % 3]],
                          ssems[lb]).wait()
    plsc.subcore_barrier()
    pltpu.sync_copy(acc.at[pl.ds(s * RPT, RPT)],
                    part_hbm.at[c, pl.ds(s * RPT, RPT)])


@functools.lru_cache(maxsize=None)
def _sc_kernels():
    mesh = plsc.VectorSubcoreMesh(core_axis_name="c", subcore_axis_name="s",
                                  num_cores=NC, num_subcores=NS)
    deg = pl.kernel(
        _deg_body,
        out_type=jax.ShapeDtypeStruct((NC, NPAD), jnp.float32),
        mesh=mesh,
        scratch_types=[
            pltpu.VMEM((CH,), jnp.int32),
            pltpu.VMEM((CH,), jnp.int32),
            pltpu.VMEM((CH,), jnp.float32),
            pltpu.VMEM_SHARED((NPAD,), jnp.float32),
            pltpu.SemaphoreType.DMA,
            pltpu.SemaphoreType.DMA,
            pltpu.SemaphoreType.DMA,
            pltpu.SemaphoreType.DMA,
        ],
    )

    scat = pl.kernel(
        _scat_body,
        out_type=jax.ShapeDtypeStruct((NC, NPAD, H2), jnp.float32),
        mesh=mesh,
        scratch_types=[
            pltpu.VMEM((CH,), jnp.int32),
            pltpu.VMEM((CH,), jnp.int32),
            pltpu.VMEM((CH,), jnp.int32),
            pltpu.VMEM((CH,), jnp.int32),
            pltpu.VMEM((CH,), jnp.int32),
            pltpu.VMEM((CH,), jnp.int32),
            pltpu.VMEM((CH, H2), jnp.float32),
            pltpu.VMEM((CH, H2), jnp.float32),
            pltpu.VMEM_SHARED((NPAD, H2), jnp.float32),
            pltpu.SemaphoreType.DMA,
            pltpu.SemaphoreType.DMA,
            pltpu.SemaphoreType.DMA,
            pltpu.SemaphoreType.DMA,
            pltpu.SemaphoreType.DMA,
            pltpu.SemaphoreType.DMA,
        ],
    )

    return deg, scat


def _dinv_of(dp_ref):
    deg = dp_ref[0, :] + dp_ref[1, :] + 1.0
    return lax.rsqrt(deg)[:, None]


def _tc1_body(x_ref, w1_ref, dp_ref, hs1_ref):
    dinv = _dinv_of(dp_ref)
    hs1 = dinv * jnp.dot(x_ref[...], w1_ref[...],
                         preferred_element_type=jnp.float32)
    hs1_ref[...] = jnp.concatenate(
        [hs1, jnp.zeros((BLK, H2 - H1), jnp.float32)], axis=1)


def _tc2_body(p1_ref, hs1_ref, dp_ref, b1_ref, w2_ref, hs2_ref):
    dinv = _dinv_of(dp_ref)
    agg = (p1_ref[0] + p1_ref[1] + hs1_ref[...])[:, :H1]
    h = jnp.maximum(dinv * agg + b1_ref[...], 0.0)
    hs2_ref[...] = dinv * jnp.dot(h, w2_ref[...],
                                  preferred_element_type=jnp.float32)


def _tc3_body(p2_ref, hs2_ref, dp_ref, b2_ref, bat_ref, psum_ref, cnt_ref):
    i = pl.program_id(0)
    dinv = _dinv_of(dp_ref)
    h2 = jnp.maximum(dinv * (p2_ref[0] + p2_ref[1] + hs2_ref[...]) + b2_ref[...],
                     0.0)
    gid = lax.broadcasted_iota(jnp.int32, (BLK, G), 1)
    oh = (bat_ref[...] == gid).astype(jnp.float32)

    @pl.when(i == 0)
    def _():
        psum_ref[...] = jnp.zeros_like(psum_ref)
        cnt_ref[...] = jnp.zeros_like(cnt_ref)

    psum_ref[...] += lax.dot_general(oh, h2, (((0,), (0,)), ((), ())),
                                     preferred_element_type=jnp.float32)
    cnt_ref[...] += lax.dot_general(oh, jnp.ones((BLK, H2), jnp.float32),
                                    (((0,), (0,)), ((), ())),
                                    preferred_element_type=jnp.float32)


def _tc4_body(psum_ref, cnt_ref, fw1_ref, fb1_ref, fw2_ref, fb2_ref, out_ref):
    pooled = psum_ref[...] / jnp.maximum(cnt_ref[...], 1.0)
    z = jnp.maximum(jnp.dot(pooled, fw1_ref[...],
                            preferred_element_type=jnp.float32) + fb1_ref[...],
                    0.0)
    out_ref[...] = jnp.dot(z, fw2_ref[...],
                           preferred_element_type=jnp.float32) + fb2_ref[...]


_tc1 = pl.pallas_call(
    _tc1_body,
    grid=(NB,),
    in_specs=[
        pl.BlockSpec((BLK, IN), lambda i: (i, 0)),
        pl.BlockSpec((IN, H1), lambda i: (0, 0)),
        pl.BlockSpec((2, BLK), lambda i: (0, i)),
    ],
    out_specs=pl.BlockSpec((BLK, H2), lambda i: (i, 0)),
    out_shape=jax.ShapeDtypeStruct((NPAD, H2), jnp.float32),
)

_tc2 = pl.pallas_call(
    _tc2_body,
    grid=(NB,),
    in_specs=[
        pl.BlockSpec((2, BLK, H2), lambda i: (0, i, 0)),
        pl.BlockSpec((BLK, H2), lambda i: (i, 0)),
        pl.BlockSpec((2, BLK), lambda i: (0, i)),
        pl.BlockSpec((1, H1), lambda i: (0, 0)),
        pl.BlockSpec((H1, H2), lambda i: (0, 0)),
    ],
    out_specs=pl.BlockSpec((BLK, H2), lambda i: (i, 0)),
    out_shape=jax.ShapeDtypeStruct((NPAD, H2), jnp.float32),
)

_tc3 = pl.pallas_call(
    _tc3_body,
    grid=(NB,),
    in_specs=[
        pl.BlockSpec((2, BLK, H2), lambda i: (0, i, 0)),
        pl.BlockSpec((BLK, H2), lambda i: (i, 0)),
        pl.BlockSpec((2, BLK), lambda i: (0, i)),
        pl.BlockSpec((1, H2), lambda i: (0, 0)),
        pl.BlockSpec((BLK, 1), lambda i: (i, 0)),
    ],
    out_specs=[
        pl.BlockSpec((G, H2), lambda i: (0, 0)),
        pl.BlockSpec((G, H2), lambda i: (0, 0)),
    ],
    out_shape=[
        jax.ShapeDtypeStruct((G, H2), jnp.float32),
        jax.ShapeDtypeStruct((G, H2), jnp.float32),
    ],
)

_OB = OUT // 10
_tc4 = pl.pallas_call(
    _tc4_body,
    grid=(10,),
    in_specs=[
        pl.BlockSpec((G, H2), lambda i: (0, 0)),
        pl.BlockSpec((G, H2), lambda i: (0, 0)),
        pl.BlockSpec((H2, FC1), lambda i: (0, 0)),
        pl.BlockSpec((1, FC1), lambda i: (0, 0)),
        pl.BlockSpec((FC1, _OB), lambda i: (0, i)),
        pl.BlockSpec((1, _OB), lambda i: (0, i)),
    ],
    out_specs=pl.BlockSpec((G, _OB), lambda i: (0, i)),
    out_shape=jax.ShapeDtypeStruct((G, OUT), jnp.float32),
)


def kernel(x, edge_index, batch, W1, b1, W2, b2, fcW1, fcb1, fcW2, fcb2):
    x_p = jnp.pad(x, ((0, NPAD - N), (0, 0)))
    bat_p = jnp.pad(batch, (0, NPAD - N), constant_values=G).reshape(NPAD, 1)
    zd = jnp.zeros((NPAD,), jnp.float32)
    z2 = jnp.zeros((NPAD, H2), jnp.float32)

    src = edge_index[0]
    dst = edge_index[1]
    _deg, _scat = _sc_kernels()
    dp = _deg(dst, zd)
    hs1 = _tc1(x_p, W1, dp)
    p1 = _scat(hs1, src, dst, z2)
    hs2 = _tc2(p1, hs1, dp, b1.reshape(1, H1), W2)
    p2 = _scat(hs2, src, dst, z2)
    psum, cnt = _tc3(p2, hs2, dp, b2.reshape(1, H2), bat_p)
    out = _tc4(psum, cnt, fcW1, fcb1.reshape(1, FC1), fcW2, fcb2.reshape(1, OUT))
    return out


# speedup vs baseline: 23.7893x; 1.1191x over previous
import functools

import jax
import jax.numpy as jnp
from jax import lax
from jax.experimental import pallas as pl
from jax.experimental.pallas import tpu as pltpu
from jax.experimental.pallas import tpu_sc as plsc

N = 10000
E = 320000
IN = 128
H1 = 64
H2 = 128
FC1 = 1024
OUT = 6400
G = 64

NC, NS = 2, 16
NW = NC * NS
NPAD = 10240
RPT = NPAD // NS
EW = E // NW
CH = 80
NCHUNK = EW // CH

BLK = 512
NB = NPAD // BLK

def _deg_body(dst_hbm, zd_hbm, dp_hbm, di0, di1, di2, ones_v, acc,
              ssem0, ssem1, isem0, isem1, isem2):
    c = lax.axis_index("c")
    s = lax.axis_index("s")
    wid = s * NC + c
    pltpu.sync_copy(zd_hbm.at[pl.ds(s * RPT, RPT)], acc.at[pl.ds(s * RPT, RPT)])
    for i in range(CH // 16):
        ones_v[pl.ds(i * 16, 16)] = jnp.full((16,), 1.0, jnp.float32)
    plsc.subcore_barrier()
    base = wid * EW
    dib = (di0, di1, di2)
    ssems = (ssem0, ssem1)
    isems = (isem0, isem1, isem2)

    def idx_issue(t, k3):
        pltpu.async_copy(dst_hbm.at[pl.ds(base + t * CH, CH)], dib[k3], isems[k3])

    def idx_wait(t, k3):
        pltpu.make_async_copy(dst_hbm.at[pl.ds(base + t * CH, CH)],
                              dib[k3], isems[k3]).wait()

    def substep(t, k2, k3, first=False, idx_next=True):
        idx_wait(t, k3)
        pltpu.async_copy(ones_v, acc.at[dib[k3]], ssems[k2], add=True)
        if not first:
            pltpu.make_async_copy(ones_v, acc.at[dib[(k3 + 2) % 3]],
                                  ssems[1 - k2]).wait()
        if idx_next:
            idx_issue(t + 2, (k3 + 2) % 3)

    idx_issue(0, 0)
    idx_issue(1, 1)
    substep(0, 0, 0, first=True)
    for t in range(1, 6):
        substep(t, t % 2, t % 3)

    def step(jj, carry):
        t = 6 * jj
        for k in range(6):
            substep(t + k, k % 2, k % 3)
        return carry

    lax.fori_loop(1, (NCHUNK - 5) // 6, step, 0)
    for t in range(NCHUNK - 5, NCHUNK):
        substep(t, t % 2, t % 3, idx_next=(t + 2 < NCHUNK))
    lb = (NCHUNK - 1) % 2
    pltpu.make_async_copy(ones_v, acc.at[dib[(NCHUNK - 1) % 3]],
                          ssems[lb]).wait()
    plsc.subcore_barrier()
    pltpu.sync_copy(acc.at[pl.ds(s * RPT, RPT)], dp_hbm.at[c, pl.ds(s * RPT, RPT)])


def _scat_body(hs_hbm, src_hbm, dst_hbm, zr_hbm, part_hbm,
               si0, si1, si2, di0, di1, di2, rows0, rows1, acc,
               gsem, ssem0, ssem1, isem0, isem1, isem2):
    c = lax.axis_index("c")
    s = lax.axis_index("s")
    wid = s * NC + c
    pltpu.sync_copy(zr_hbm.at[pl.ds(s * RPT, RPT)], acc.at[pl.ds(s * RPT, RPT)])
    plsc.subcore_barrier()
    base = wid * EW
    sib = (si0, si1, si2)
    dib = (di0, di1, di2)
    rbufs = (rows0, rows1)
    ssems = (ssem0, ssem1)
    isems = (isem0, isem1, isem2)

    def idx_issue(t, k3):
        off = base + t * CH
        pltpu.async_copy(src_hbm.at[pl.ds(off, CH)], sib[k3], isems[k3])
        pltpu.async_copy(dst_hbm.at[pl.ds(off, CH)], dib[k3], isems[k3])

    def idx_wait(t, k3):
        off = base + t * CH
        pltpu.make_async_copy(src_hbm.at[pl.ds(off, CH)], sib[k3], isems[k3]).wait()
        pltpu.make_async_copy(dst_hbm.at[pl.ds(off, CH)], dib[k3], isems[k3]).wait()

    def substep(t, k2, k3, first=False, gather_next=True, idx_next=True):
        rc, ro = rbufs[k2], rbufs[1 - k2]
        pltpu.make_async_copy(hs_hbm.at[sib[k3]], rc, gsem).wait()
        pltpu.async_copy(rc, acc.at[dib[k3]], ssems[k2], add=True)
        if not first:
            pltpu.make_async_copy(ro, acc.at[dib[(k3 + 2) % 3]],
                                  ssems[1 - k2]).wait()
        if gather_next:
            idx_wait(t + 1, (k3 + 1) % 3)
            pltpu.async_copy(hs_hbm.at[sib[(k3 + 1) % 3]], ro, gsem)
        if idx_next:
            idx_issue(t + 2, (k3 + 2) % 3)

    idx_issue(0, 0)
    idx_issue(1, 1)
    idx_wait(0, 0)
    pltpu.async_copy(hs_hbm.at[si0], rows0, gsem)
    substep(0, 0, 0, first=True)
    for t in range(1, 6):
        substep(t, t % 2, t % 3)

    def step(jj, carry):
        t = 6 * jj
        for k in range(6):
            substep(t + k, k % 2, k % 3)
        return carry

    lax.fori_loop(1, (NCHUNK - 5) // 6, step, 0)
    for t in range(NCHUNK - 5, NCHUNK):
        substep(t, t % 2, t % 3,
                gather_next=(t + 1 < NCHUNK), idx_next=(t + 2 < NCHUNK))
    lb = (NCHUNK - 1) % 2
    pltpu.make_async_copy(rbufs[lb], acc.at[dib[(NCHUNK - 1) % 3]],
                          ssems[lb]).wait()
    plsc.subcore_barrier()
    pltpu.sync_copy(acc.at[pl.ds(s * RPT, RPT)],
                    part_hbm.at[c, pl.ds(s * RPT, RPT)])


@functools.lru_cache(maxsize=None)
def _sc_kernels():
    mesh = plsc.VectorSubcoreMesh(core_axis_name="c", subcore_axis_name="s",
                                  num_cores=NC, num_subcores=NS)
    deg = pl.kernel(
        _deg_body,
        out_type=jax.ShapeDtypeStruct((NC, NPAD), jnp.float32),
        mesh=mesh,
        scratch_types=[
            pltpu.VMEM((CH,), jnp.int32),
            pltpu.VMEM((CH,), jnp.int32),
            pltpu.VMEM((CH,), jnp.int32),
            pltpu.VMEM((CH,), jnp.float32),
            pltpu.VMEM_SHARED((NPAD,), jnp.float32),
            pltpu.SemaphoreType.DMA,
            pltpu.SemaphoreType.DMA,
            pltpu.SemaphoreType.DMA,
            pltpu.SemaphoreType.DMA,
            pltpu.SemaphoreType.DMA,
        ],
    )

    def make_scat(H, untiled):
        params = pltpu.CompilerParams(use_tc_tiling_on_sc=False) if untiled else None
        return pl.kernel(
            _scat_body,
            out_type=jax.ShapeDtypeStruct((NC, NPAD, H), jnp.float32),
            mesh=mesh,
            compiler_params=params,
            scratch_types=[
            pltpu.VMEM((CH,), jnp.int32),
            pltpu.VMEM((CH,), jnp.int32),
            pltpu.VMEM((CH,), jnp.int32),
            pltpu.VMEM((CH,), jnp.int32),
            pltpu.VMEM((CH,), jnp.int32),
            pltpu.VMEM((CH,), jnp.int32),
            pltpu.VMEM((CH, H), jnp.float32),
            pltpu.VMEM((CH, H), jnp.float32),
            pltpu.VMEM_SHARED((NPAD, H), jnp.float32),
            pltpu.SemaphoreType.DMA,
            pltpu.SemaphoreType.DMA,
            pltpu.SemaphoreType.DMA,
            pltpu.SemaphoreType.DMA,
            pltpu.SemaphoreType.DMA,
            pltpu.SemaphoreType.DMA,
        ],
        )

    return deg, make_scat(H1, True), make_scat(H2, False)


def _dinv_of(dp_ref):
    deg = dp_ref[0, :] + dp_ref[1, :] + 1.0
    return lax.rsqrt(deg)[:, None]


def _tc1_body(x_ref, w1_ref, dp_ref, hs1_ref):
    dinv = _dinv_of(dp_ref)
    hs1_ref[...] = dinv * jnp.dot(x_ref[...], w1_ref[...],
                                  preferred_element_type=jnp.float32)


def _tc2_body(p1_ref, hs1_ref, dp_ref, b1_ref, w2_ref, hs2_ref):
    dinv = _dinv_of(dp_ref)
    agg = p1_ref[0] + p1_ref[1] + hs1_ref[...]
    h = jnp.maximum(dinv * agg + b1_ref[...], 0.0)
    hs2_ref[...] = dinv * jnp.dot(h, w2_ref[...],
                                  preferred_element_type=jnp.float32)


def _tc3_body(p2_ref, hs2_ref, dp_ref, b2_ref, bat_ref, psum_ref, cnt_ref):
    i = pl.program_id(0)
    dinv = _dinv_of(dp_ref)
    h2 = jnp.maximum(dinv * (p2_ref[0] + p2_ref[1] + hs2_ref[...]) + b2_ref[...],
                     0.0)
    gid = lax.broadcasted_iota(jnp.int32, (BLK, G), 1)
    oh = (bat_ref[...] == gid).astype(jnp.float32)

    @pl.when(i == 0)
    def _():
        psum_ref[...] = jnp.zeros_like(psum_ref)
        cnt_ref[...] = jnp.zeros_like(cnt_ref)

    psum_ref[...] += lax.dot_general(oh, h2, (((0,), (0,)), ((), ())),
                                     preferred_element_type=jnp.float32)
    cnt_ref[...] += lax.dot_general(oh, jnp.ones((BLK, H2), jnp.float32),
                                    (((0,), (0,)), ((), ())),
                                    preferred_element_type=jnp.float32)


def _tc4_body(psum_ref, cnt_ref, fw1_ref, fb1_ref, fw2_ref, fb2_ref, out_ref):
    pooled = psum_ref[...] / jnp.maximum(cnt_ref[...], 1.0)
    z = jnp.maximum(jnp.dot(pooled, fw1_ref[...],
                            preferred_element_type=jnp.float32) + fb1_ref[...],
                    0.0)
    out_ref[...] = jnp.dot(z, fw2_ref[...],
                           preferred_element_type=jnp.float32) + fb2_ref[...]


_tc1 = pl.pallas_call(
    _tc1_body,
    grid=(NB,),
    in_specs=[
        pl.BlockSpec((BLK, IN), lambda i: (i, 0)),
        pl.BlockSpec((IN, H1), lambda i: (0, 0)),
        pl.BlockSpec((2, BLK), lambda i: (0, i)),
    ],
    out_specs=pl.BlockSpec((BLK, H1), lambda i: (i, 0)),
    out_shape=jax.ShapeDtypeStruct((NPAD, H1), jnp.float32),
)

_tc2 = pl.pallas_call(
    _tc2_body,
    grid=(NB,),
    in_specs=[
        pl.BlockSpec((2, BLK, H1), lambda i: (0, i, 0)),
        pl.BlockSpec((BLK, H1), lambda i: (i, 0)),
        pl.BlockSpec((2, BLK), lambda i: (0, i)),
        pl.BlockSpec((1, H1), lambda i: (0, 0)),
        pl.BlockSpec((H1, H2), lambda i: (0, 0)),
    ],
    out_specs=pl.BlockSpec((BLK, H2), lambda i: (i, 0)),
    out_shape=jax.ShapeDtypeStruct((NPAD, H2), jnp.float32),
)

_tc3 = pl.pallas_call(
    _tc3_body,
    grid=(NB,),
    in_specs=[
        pl.BlockSpec((2, BLK, H2), lambda i: (0, i, 0)),
        pl.BlockSpec((BLK, H2), lambda i: (i, 0)),
        pl.BlockSpec((2, BLK), lambda i: (0, i)),
        pl.BlockSpec((1, H2), lambda i: (0, 0)),
        pl.BlockSpec((BLK, 1), lambda i: (i, 0)),
    ],
    out_specs=[
        pl.BlockSpec((G, H2), lambda i: (0, 0)),
        pl.BlockSpec((G, H2), lambda i: (0, 0)),
    ],
    out_shape=[
        jax.ShapeDtypeStruct((G, H2), jnp.float32),
        jax.ShapeDtypeStruct((G, H2), jnp.float32),
    ],
)

_OB = OUT // 10
_tc4 = pl.pallas_call(
    _tc4_body,
    grid=(10,),
    in_specs=[
        pl.BlockSpec((G, H2), lambda i: (0, 0)),
        pl.BlockSpec((G, H2), lambda i: (0, 0)),
        pl.BlockSpec((H2, FC1), lambda i: (0, 0)),
        pl.BlockSpec((1, FC1), lambda i: (0, 0)),
        pl.BlockSpec((FC1, _OB), lambda i: (0, i)),
        pl.BlockSpec((1, _OB), lambda i: (0, i)),
    ],
    out_specs=pl.BlockSpec((G, _OB), lambda i: (0, i)),
    out_shape=jax.ShapeDtypeStruct((G, OUT), jnp.float32),
)


def kernel(x, edge_index, batch, W1, b1, W2, b2, fcW1, fcb1, fcW2, fcb2):
    x_p = jnp.pad(x, ((0, NPAD - N), (0, 0)))
    bat_p = jnp.pad(batch, (0, NPAD - N), constant_values=G).reshape(NPAD, 1)
    zd = jnp.zeros((NPAD,), jnp.float32)
    z1 = jnp.zeros((NPAD, H1), jnp.float32)
    z2 = jnp.zeros((NPAD, H2), jnp.float32)

    src = edge_index[0]
    dst = edge_index[1]
    _deg, _scat1, _scat2 = _sc_kernels()
    dp = _deg(dst, zd)
    hs1 = _tc1(x_p, W1, dp)
    p1 = _scat1(hs1, src, dst, z1)
    hs2 = _tc2(p1, hs1, dp, b1.reshape(1, H1), W2)
    p2 = _scat2(hs2, src, dst, z2)
    psum, cnt = _tc3(p2, hs2, dp, b2.reshape(1, H2), bat_p)
    out = _tc4(psum, cnt, fcW1, fcb1.reshape(1, FC1), fcW2, fcb2.reshape(1, OUT))
    return out


# speedup vs baseline: 27.8014x; 1.1687x over previous
import functools

import jax
import jax.numpy as jnp
from jax import lax
from jax.experimental import pallas as pl
from jax.experimental.pallas import tpu as pltpu
from jax.experimental.pallas import tpu_sc as plsc

N = 10000
E = 320000
IN = 128
H1 = 64
H2 = 128
FC1 = 1024
OUT = 6400
G = 64

NC, NS = 2, 16
NW = NC * NS
NPAD = 10240
RPT = NPAD // NS
EWP = 10240
E_PAD = NW * EWP
CH = 128
NCHUNK = EWP // CH
_MID = (NCHUNK - 6) // 6
_TAIL = 6 + 6 * _MID

BLK = 512
NB = NPAD // BLK

def _deg_body(dst_hbm, zd_hbm, dp_hbm, di0, di1, di2, ones_v, acc,
              ssem0, ssem1, isem0, isem1, isem2):
    c = lax.axis_index("c")
    s = lax.axis_index("s")
    wid = s * NC + c
    pltpu.sync_copy(zd_hbm.at[pl.ds(s * RPT, RPT)], acc.at[pl.ds(s * RPT, RPT)])
    for i in range(CH // 16):
        ones_v[pl.ds(i * 16, 16)] = jnp.full((16,), 1.0, jnp.float32)
    plsc.subcore_barrier()
    base = wid * EWP
    dib = (di0, di1, di2)
    ssems = (ssem0, ssem1)
    isems = (isem0, isem1, isem2)

    def idx_issue(t, k3):
        pltpu.async_copy(dst_hbm.at[pl.ds(base + t * CH, CH)], dib[k3], isems[k3])

    def idx_wait(t, k3):
        pltpu.make_async_copy(dst_hbm.at[pl.ds(base + t * CH, CH)],
                              dib[k3], isems[k3]).wait()

    def substep(t, k2, k3, first=False, idx_next=True):
        idx_wait(t, k3)
        pltpu.async_copy(ones_v, acc.at[dib[k3]], ssems[k2], add=True)
        if not first:
            pltpu.make_async_copy(ones_v, acc.at[dib[(k3 + 2) % 3]],
                                  ssems[1 - k2]).wait()
        if idx_next:
            idx_issue(t + 2, (k3 + 2) % 3)

    idx_issue(0, 0)
    idx_issue(1, 1)
    substep(0, 0, 0, first=True)
    for t in range(1, 6):
        substep(t, t % 2, t % 3)

    def step(jj, carry):
        t = 6 * jj
        for k in range(6):
            substep(t + k, k % 2, k % 3)
        return carry

    lax.fori_loop(1, _MID + 1, step, 0)
    for t in range(_TAIL, NCHUNK):
        substep(t, t % 2, t % 3, idx_next=(t + 2 < NCHUNK))
    lb = (NCHUNK - 1) % 2
    pltpu.make_async_copy(ones_v, acc.at[dib[(NCHUNK - 1) % 3]],
                          ssems[lb]).wait()
    plsc.subcore_barrier()
    pltpu.sync_copy(acc.at[pl.ds(s * RPT, RPT)], dp_hbm.at[c, pl.ds(s * RPT, RPT)])


def _scat_body(hs_hbm, src_hbm, dst_hbm, zr_hbm, part_hbm,
               si0, si1, si2, di0, di1, di2, rows0, rows1, acc,
               gsem, ssem0, ssem1, isem0, isem1, isem2):
    c = lax.axis_index("c")
    s = lax.axis_index("s")
    wid = s * NC + c
    pltpu.sync_copy(zr_hbm.at[pl.ds(s * RPT, RPT)], acc.at[pl.ds(s * RPT, RPT)])
    plsc.subcore_barrier()
    base = wid * EWP
    sib = (si0, si1, si2)
    dib = (di0, di1, di2)
    rbufs = (rows0, rows1)
    ssems = (ssem0, ssem1)
    isems = (isem0, isem1, isem2)

    def idx_issue(t, k3):
        off = base + t * CH
        pltpu.async_copy(src_hbm.at[pl.ds(off, CH)], sib[k3], isems[k3])
        pltpu.async_copy(dst_hbm.at[pl.ds(off, CH)], dib[k3], isems[k3])

    def idx_wait(t, k3):
        off = base + t * CH
        pltpu.make_async_copy(src_hbm.at[pl.ds(off, CH)], sib[k3], isems[k3]).wait()
        pltpu.make_async_copy(dst_hbm.at[pl.ds(off, CH)], dib[k3], isems[k3]).wait()

    def substep(t, k2, k3, first=False, gather_next=True, idx_next=True):
        rc, ro = rbufs[k2], rbufs[1 - k2]
        pltpu.make_async_copy(hs_hbm.at[sib[k3]], rc, gsem).wait()
        pltpu.async_copy(rc, acc.at[dib[k3]], ssems[k2], add=True)
        if not first:
            pltpu.make_async_copy(ro, acc.at[dib[(k3 + 2) % 3]],
                                  ssems[1 - k2]).wait()
        if gather_next:
            idx_wait(t + 1, (k3 + 1) % 3)
            pltpu.async_copy(hs_hbm.at[sib[(k3 + 1) % 3]], ro, gsem)
        if idx_next:
            idx_issue(t + 2, (k3 + 2) % 3)

    idx_issue(0, 0)
    idx_issue(1, 1)
    idx_wait(0, 0)
    pltpu.async_copy(hs_hbm.at[si0], rows0, gsem)
    substep(0, 0, 0, first=True)
    for t in range(1, 6):
        substep(t, t % 2, t % 3)

    def step(jj, carry):
        t = 6 * jj
        for k in range(6):
            substep(t + k, k % 2, k % 3)
        return carry

    lax.fori_loop(1, _MID + 1, step, 0)
    for t in range(_TAIL, NCHUNK):
        substep(t, t % 2, t % 3,
                gather_next=(t + 1 < NCHUNK), idx_next=(t + 2 < NCHUNK))
    lb = (NCHUNK - 1) % 2
    pltpu.make_async_copy(rbufs[lb], acc.at[dib[(NCHUNK - 1) % 3]],
                          ssems[lb]).wait()
    plsc.subcore_barrier()
    pltpu.sync_copy(acc.at[pl.ds(s * RPT, RPT)],
                    part_hbm.at[c, pl.ds(s * RPT, RPT)])


@functools.lru_cache(maxsize=None)
def _sc_kernels():
    mesh = plsc.VectorSubcoreMesh(core_axis_name="c", subcore_axis_name="s",
                                  num_cores=NC, num_subcores=NS)
    deg = pl.kernel(
        _deg_body,
        out_type=jax.ShapeDtypeStruct((NC, NPAD), jnp.float32),
        mesh=mesh,
        scratch_types=[
            pltpu.VMEM((CH,), jnp.int32),
            pltpu.VMEM((CH,), jnp.int32),
            pltpu.VMEM((CH,), jnp.int32),
            pltpu.VMEM((CH,), jnp.float32),
            pltpu.VMEM_SHARED((NPAD,), jnp.float32),
            pltpu.SemaphoreType.DMA,
            pltpu.SemaphoreType.DMA,
            pltpu.SemaphoreType.DMA,
            pltpu.SemaphoreType.DMA,
            pltpu.SemaphoreType.DMA,
        ],
    )

    def make_scat(H, untiled):
        params = pltpu.CompilerParams(use_tc_tiling_on_sc=False) if untiled else None
        return pl.kernel(
            _scat_body,
            out_type=jax.ShapeDtypeStruct((NC, NPAD, H), jnp.float32),
            mesh=mesh,
            compiler_params=params,
            scratch_types=[
            pltpu.VMEM((CH,), jnp.int32),
            pltpu.VMEM((CH,), jnp.int32),
            pltpu.VMEM((CH,), jnp.int32),
            pltpu.VMEM((CH,), jnp.int32),
            pltpu.VMEM((CH,), jnp.int32),
            pltpu.VMEM((CH,), jnp.int32),
            pltpu.VMEM((CH, H), jnp.float32),
            pltpu.VMEM((CH, H), jnp.float32),
            pltpu.VMEM_SHARED((NPAD, H), jnp.float32),
            pltpu.SemaphoreType.DMA,
            pltpu.SemaphoreType.DMA,
            pltpu.SemaphoreType.DMA,
            pltpu.SemaphoreType.DMA,
            pltpu.SemaphoreType.DMA,
            pltpu.SemaphoreType.DMA,
        ],
        )

    return deg, make_scat(H1, True), make_scat(H2, False)


def _dinv_of(dp_ref):
    deg = dp_ref[0, :] + dp_ref[1, :] + 1.0
    return lax.rsqrt(deg)[:, None]


def _tc1_body(x_ref, w1_ref, dp_ref, hs1_ref):
    dinv = _dinv_of(dp_ref)
    hs1_ref[...] = dinv * jnp.dot(x_ref[...], w1_ref[...],
                                  preferred_element_type=jnp.float32)


def _tc2_body(p1_ref, hs1_ref, dp_ref, b1_ref, w2_ref, hs2_ref):
    dinv = _dinv_of(dp_ref)
    agg = p1_ref[0] + p1_ref[1] + hs1_ref[...]
    h = jnp.maximum(dinv * agg + b1_ref[...], 0.0)
    hs2 = dinv * jnp.dot(h, w2_ref[...], preferred_element_type=jnp.float32)
    row = pl.program_id(0) * BLK + lax.broadcasted_iota(jnp.int32, (BLK, 1), 0)
    hs2_ref[...] = jnp.where(row < N, hs2, 0.0)


def _tc3_body(p2_ref, hs2_ref, dp_ref, b2_ref, bat_ref, psum_ref, cnt_ref):
    i = pl.program_id(0)
    dinv = _dinv_of(dp_ref)
    h2 = jnp.maximum(dinv * (p2_ref[0] + p2_ref[1] + hs2_ref[...]) + b2_ref[...],
                     0.0)
    gid = lax.broadcasted_iota(jnp.int32, (BLK, G), 1)
    oh = (bat_ref[...] == gid).astype(jnp.float32)

    @pl.when(i == 0)
    def _():
        psum_ref[...] = jnp.zeros_like(psum_ref)
        cnt_ref[...] = jnp.zeros_like(cnt_ref)

    psum_ref[...] += lax.dot_general(oh, h2, (((0,), (0,)), ((), ())),
                                     preferred_element_type=jnp.float32)
    cnt_ref[...] += lax.dot_general(oh, jnp.ones((BLK, H2), jnp.float32),
                                    (((0,), (0,)), ((), ())),
                                    preferred_element_type=jnp.float32)


def _tc4_body(psum_ref, cnt_ref, fw1_ref, fb1_ref, fw2_ref, fb2_ref, out_ref):
    pooled = psum_ref[...] / jnp.maximum(cnt_ref[...], 1.0)
    z = jnp.maximum(jnp.dot(pooled, fw1_ref[...],
                            preferred_element_type=jnp.float32) + fb1_ref[...],
                    0.0)
    out_ref[...] = jnp.dot(z, fw2_ref[...],
                           preferred_element_type=jnp.float32) + fb2_ref[...]


_tc1 = pl.pallas_call(
    _tc1_body,
    grid=(NB,),
    in_specs=[
        pl.BlockSpec((BLK, IN), lambda i: (i, 0)),
        pl.BlockSpec((IN, H1), lambda i: (0, 0)),
        pl.BlockSpec((2, BLK), lambda i: (0, i)),
    ],
    out_specs=pl.BlockSpec((BLK, H1), lambda i: (i, 0)),
    out_shape=jax.ShapeDtypeStruct((NPAD, H1), jnp.float32),
)

_tc2 = pl.pallas_call(
    _tc2_body,
    grid=(NB,),
    in_specs=[
        pl.BlockSpec((2, BLK, H1), lambda i: (0, i, 0)),
        pl.BlockSpec((BLK, H1), lambda i: (i, 0)),
        pl.BlockSpec((2, BLK), lambda i: (0, i)),
        pl.BlockSpec((1, H1), lambda i: (0, 0)),
        pl.BlockSpec((H1, H2), lambda i: (0, 0)),
    ],
    out_specs=pl.BlockSpec((BLK, H2), lambda i: (i, 0)),
    out_shape=jax.ShapeDtypeStruct((NPAD, H2), jnp.float32),
)

_tc3 = pl.pallas_call(
    _tc3_body,
    grid=(NB,),
    in_specs=[
        pl.BlockSpec((2, BLK, H2), lambda i: (0, i, 0)),
        pl.BlockSpec((BLK, H2), lambda i: (i, 0)),
        pl.BlockSpec((2, BLK), lambda i: (0, i)),
        pl.BlockSpec((1, H2), lambda i: (0, 0)),
        pl.BlockSpec((BLK, 1), lambda i: (i, 0)),
    ],
    out_specs=[
        pl.BlockSpec((G, H2), lambda i: (0, 0)),
        pl.BlockSpec((G, H2), lambda i: (0, 0)),
    ],
    out_shape=[
        jax.ShapeDtypeStruct((G, H2), jnp.float32),
        jax.ShapeDtypeStruct((G, H2), jnp.float32),
    ],
)

_OB = OUT // 10
_tc4 = pl.pallas_call(
    _tc4_body,
    grid=(10,),
    in_specs=[
        pl.BlockSpec((G, H2), lambda i: (0, 0)),
        pl.BlockSpec((G, H2), lambda i: (0, 0)),
        pl.BlockSpec((H2, FC1), lambda i: (0, 0)),
        pl.BlockSpec((1, FC1), lambda i: (0, 0)),
        pl.BlockSpec((FC1, _OB), lambda i: (0, i)),
        pl.BlockSpec((1, _OB), lambda i: (0, i)),
    ],
    out_specs=pl.BlockSpec((G, _OB), lambda i: (0, i)),
    out_shape=jax.ShapeDtypeStruct((G, OUT), jnp.float32),
)


def kernel(x, edge_index, batch, W1, b1, W2, b2, fcW1, fcb1, fcW2, fcb2):
    x_p = jnp.pad(x, ((0, NPAD - N), (0, 0)))
    bat_p = jnp.pad(batch, (0, NPAD - N), constant_values=G).reshape(NPAD, 1)
    zd = jnp.zeros((NPAD,), jnp.float32)
    z1 = jnp.zeros((NPAD, H1), jnp.float32)
    z2 = jnp.zeros((NPAD, H2), jnp.float32)

    pad_idx = N + (jnp.arange(E_PAD - E, dtype=jnp.int32) % (NPAD - N))
    src = jnp.concatenate([edge_index[0], pad_idx])
    dst = jnp.concatenate([edge_index[1], pad_idx])
    _deg, _scat1, _scat2 = _sc_kernels()
    dp = _deg(dst, zd)
    hs1 = _tc1(x_p, W1, dp)
    p1 = _scat1(hs1, src, dst, z1)
    hs2 = _tc2(p1, hs1, dp, b1.reshape(1, H1), W2)
    p2 = _scat2(hs2, src, dst, z2)
    psum, cnt = _tc3(p2, hs2, dp, b2.reshape(1, H2), bat_p)
    out = _tc4(psum, cnt, fcW1, fcb1.reshape(1, FC1), fcW2, fcb2.reshape(1, OUT))
    return out


# speedup vs baseline: 28.2683x; 1.0168x over previous
import functools

import jax
import jax.numpy as jnp
from jax import lax
from jax.experimental import pallas as pl
from jax.experimental.pallas import tpu as pltpu
from jax.experimental.pallas import tpu_sc as plsc

N = 10000
E = 320000
IN = 128
H1 = 64
H2 = 128
FC1 = 1024
OUT = 6400
G = 64

NC, NS = 2, 16
NW = NC * NS
NPAD = 10240
RPT = NPAD // NS
EWP = 10240
E_PAD = NW * EWP
CH = 128
NCHUNK = EWP // CH
_MID = (NCHUNK - 6) // 6
_TAIL = 6 + 6 * _MID

BLK = 512
NB = NPAD // BLK

def _deg_body(dst_hbm, zd_hbm, dp_hbm, di0, di1, di2, ones_v, acc,
              ssem0, ssem1, isem0, isem1, isem2):
    c = lax.axis_index("c")
    s = lax.axis_index("s")
    wid = s * NC + c
    pltpu.sync_copy(zd_hbm.at[pl.ds(s * RPT, RPT)], acc.at[pl.ds(s * RPT, RPT)])
    for i in range(CH // 16):
        ones_v[pl.ds(i * 16, 16)] = jnp.full((16,), 1.0, jnp.float32)
    plsc.subcore_barrier()
    base = wid * EWP
    dib = (di0, di1, di2)
    ssems = (ssem0, ssem1)
    isems = (isem0, isem1, isem2)

    def idx_issue(t, k3):
        pltpu.async_copy(dst_hbm.at[pl.ds(base + t * CH, CH)], dib[k3], isems[k3])

    def idx_wait(t, k3):
        pltpu.make_async_copy(dst_hbm.at[pl.ds(base + t * CH, CH)],
                              dib[k3], isems[k3]).wait()

    def substep(t, k2, k3, first=False, idx_next=True):
        idx_wait(t, k3)
        pltpu.async_copy(ones_v, acc.at[dib[k3]], ssems[k2], add=True)
        if not first:
            pltpu.make_async_copy(ones_v, acc.at[dib[(k3 + 2) % 3]],
                                  ssems[1 - k2]).wait()
        if idx_next:
            idx_issue(t + 2, (k3 + 2) % 3)

    idx_issue(0, 0)
    idx_issue(1, 1)
    substep(0, 0, 0, first=True)
    for t in range(1, 6):
        substep(t, t % 2, t % 3)

    def step(jj, carry):
        t = 6 * jj
        for k in range(6):
            substep(t + k, k % 2, k % 3)
        return carry

    lax.fori_loop(1, _MID + 1, step, 0)
    for t in range(_TAIL, NCHUNK):
        substep(t, t % 2, t % 3, idx_next=(t + 2 < NCHUNK))
    lb = (NCHUNK - 1) % 2
    pltpu.make_async_copy(ones_v, acc.at[dib[(NCHUNK - 1) % 3]],
                          ssems[lb]).wait()
    plsc.subcore_barrier()
    pltpu.sync_copy(acc.at[pl.ds(s * RPT, RPT)], dp_hbm.at[c, pl.ds(s * RPT, RPT)])


def _scat_body(hs_hbm, ei_hbm, zr_hbm, part_hbm,
               ib0, ib1, ib2, rows0, rows1, acc,
               gsem, ssem0, ssem1, isem0, isem1, isem2):
    c = lax.axis_index("c")
    s = lax.axis_index("s")
    wid = s * NC + c
    pltpu.sync_copy(zr_hbm.at[pl.ds(s * RPT, RPT)], acc.at[pl.ds(s * RPT, RPT)])
    plsc.subcore_barrier()
    base = wid * EWP
    ib = (ib0, ib1, ib2)
    rbufs = (rows0, rows1)
    ssems = (ssem0, ssem1)
    isems = (isem0, isem1, isem2)

    def idx_issue(t, k3):
        pltpu.async_copy(ei_hbm.at[:, pl.ds(base + t * CH, CH)], ib[k3], isems[k3])

    def idx_wait(t, k3):
        pltpu.make_async_copy(ei_hbm.at[:, pl.ds(base + t * CH, CH)],
                              ib[k3], isems[k3]).wait()

    def substep(t, k2, k3, first=False, gather_next=True, idx_next=True):
        rc, ro = rbufs[k2], rbufs[1 - k2]
        pltpu.make_async_copy(hs_hbm.at[ib[k3].at[0]], rc, gsem).wait()
        pltpu.async_copy(rc, acc.at[ib[k3].at[1]], ssems[k2], add=True)
        if not first:
            pltpu.make_async_copy(ro, acc.at[ib[(k3 + 2) % 3].at[1]],
                                  ssems[1 - k2]).wait()
        if gather_next:
            idx_wait(t + 1, (k3 + 1) % 3)
            pltpu.async_copy(hs_hbm.at[ib[(k3 + 1) % 3].at[0]], ro, gsem)
        if idx_next:
            idx_issue(t + 2, (k3 + 2) % 3)

    idx_issue(0, 0)
    idx_issue(1, 1)
    idx_wait(0, 0)
    pltpu.async_copy(hs_hbm.at[ib0.at[0]], rows0, gsem)
    substep(0, 0, 0, first=True)
    for t in range(1, 6):
        substep(t, t % 2, t % 3)

    def step(jj, carry):
        t = 6 * jj
        for k in range(6):
            substep(t + k, k % 2, k % 3)
        return carry

    lax.fori_loop(1, _MID + 1, step, 0)
    for t in range(_TAIL, NCHUNK):
        substep(t, t % 2, t % 3,
                gather_next=(t + 1 < NCHUNK), idx_next=(t + 2 < NCHUNK))
    lb = (NCHUNK - 1) % 2
    pltpu.make_async_copy(rbufs[lb], acc.at[ib[(NCHUNK - 1) % 3].at[1]],
                          ssems[lb]).wait()
    plsc.subcore_barrier()
    pltpu.sync_copy(acc.at[pl.ds(s * RPT, RPT)],
                    part_hbm.at[c, pl.ds(s * RPT, RPT)])


@functools.lru_cache(maxsize=None)
def _sc_kernels():
    mesh = plsc.VectorSubcoreMesh(core_axis_name="c", subcore_axis_name="s",
                                  num_cores=NC, num_subcores=NS)
    deg = pl.kernel(
        _deg_body,
        out_type=jax.ShapeDtypeStruct((NC, NPAD), jnp.float32),
        mesh=mesh,
        scratch_types=[
            pltpu.VMEM((CH,), jnp.int32),
            pltpu.VMEM((CH,), jnp.int32),
            pltpu.VMEM((CH,), jnp.int32),
            pltpu.VMEM((CH,), jnp.float32),
            pltpu.VMEM_SHARED((NPAD,), jnp.float32),
            pltpu.SemaphoreType.DMA,
            pltpu.SemaphoreType.DMA,
            pltpu.SemaphoreType.DMA,
            pltpu.SemaphoreType.DMA,
            pltpu.SemaphoreType.DMA,
        ],
    )

    def make_scat(H, untiled):
        params = pltpu.CompilerParams(use_tc_tiling_on_sc=False) if untiled else None
        return pl.kernel(
            _scat_body,
            out_type=jax.ShapeDtypeStruct((NC, NPAD, H), jnp.float32),
            mesh=mesh,
            compiler_params=params,
            scratch_types=[
            pltpu.VMEM((2, CH), jnp.int32),
            pltpu.VMEM((2, CH), jnp.int32),
            pltpu.VMEM((2, CH), jnp.int32),
            pltpu.VMEM((CH, H), jnp.float32),
            pltpu.VMEM((CH, H), jnp.float32),
            pltpu.VMEM_SHARED((NPAD, H), jnp.float32),
            pltpu.SemaphoreType.DMA,
            pltpu.SemaphoreType.DMA,
            pltpu.SemaphoreType.DMA,
            pltpu.SemaphoreType.DMA,
            pltpu.SemaphoreType.DMA,
            pltpu.SemaphoreType.DMA,
        ],
        )

    return deg, make_scat(H1, True), make_scat(H2, False)


def _dinv_of(dp_ref):
    deg = dp_ref[0, :] + dp_ref[1, :] + 1.0
    return lax.rsqrt(deg)[:, None]


def _tc1_body(x_ref, w1_ref, dp_ref, hs1_ref):
    dinv = _dinv_of(dp_ref)
    hs1_ref[...] = dinv * jnp.dot(x_ref[...], w1_ref[...],
                                  preferred_element_type=jnp.float32)


def _tc2_body(p1_ref, hs1_ref, dp_ref, b1_ref, w2_ref, hs2_ref):
    dinv = _dinv_of(dp_ref)
    agg = p1_ref[0] + p1_ref[1] + hs1_ref[...]
    h = jnp.maximum(dinv * agg + b1_ref[...], 0.0)
    hs2 = dinv * jnp.dot(h, w2_ref[...], preferred_element_type=jnp.float32)
    row = pl.program_id(0) * BLK + lax.broadcasted_iota(jnp.int32, (BLK, 1), 0)
    hs2_ref[...] = jnp.where(row < N, hs2, 0.0)


def _tc3_body(p2_ref, hs2_ref, dp_ref, b2_ref, bat_ref, psum_ref, cnt_ref):
    i = pl.program_id(0)
    dinv = _dinv_of(dp_ref)
    h2 = jnp.maximum(dinv * (p2_ref[0] + p2_ref[1] + hs2_ref[...]) + b2_ref[...],
                     0.0)
    gid = lax.broadcasted_iota(jnp.int32, (BLK, G), 1)
    oh = (bat_ref[...] == gid).astype(jnp.float32)

    @pl.when(i == 0)
    def _():
        psum_ref[...] = jnp.zeros_like(psum_ref)
        cnt_ref[...] = jnp.zeros_like(cnt_ref)

    psum_ref[...] += lax.dot_general(oh, h2, (((0,), (0,)), ((), ())),
                                     preferred_element_type=jnp.float32)
    cnt_ref[...] += lax.dot_general(oh, jnp.ones((BLK, H2), jnp.float32),
                                    (((0,), (0,)), ((), ())),
                                    preferred_element_type=jnp.float32)


def _tc4_body(psum_ref, cnt_ref, fw1_ref, fb1_ref, fw2_ref, fb2_ref, out_ref):
    pooled = psum_ref[...] / jnp.maximum(cnt_ref[...], 1.0)
    z = jnp.maximum(jnp.dot(pooled, fw1_ref[...],
                            preferred_element_type=jnp.float32) + fb1_ref[...],
                    0.0)
    out_ref[...] = jnp.dot(z, fw2_ref[...],
                           preferred_element_type=jnp.float32) + fb2_ref[...]


_tc1 = pl.pallas_call(
    _tc1_body,
    grid=(NB,),
    in_specs=[
        pl.BlockSpec((BLK, IN), lambda i: (i, 0)),
        pl.BlockSpec((IN, H1), lambda i: (0, 0)),
        pl.BlockSpec((2, BLK), lambda i: (0, i)),
    ],
    out_specs=pl.BlockSpec((BLK, H1), lambda i: (i, 0)),
    out_shape=jax.ShapeDtypeStruct((NPAD, H1), jnp.float32),
)

_tc2 = pl.pallas_call(
    _tc2_body,
    grid=(NB,),
    in_specs=[
        pl.BlockSpec((2, BLK, H1), lambda i: (0, i, 0)),
        pl.BlockSpec((BLK, H1), lambda i: (i, 0)),
        pl.BlockSpec((2, BLK), lambda i: (0, i)),
        pl.BlockSpec((1, H1), lambda i: (0, 0)),
        pl.BlockSpec((H1, H2), lambda i: (0, 0)),
    ],
    out_specs=pl.BlockSpec((BLK, H2), lambda i: (i, 0)),
    out_shape=jax.ShapeDtypeStruct((NPAD, H2), jnp.float32),
)

_tc3 = pl.pallas_call(
    _tc3_body,
    grid=(NB,),
    in_specs=[
        pl.BlockSpec((2, BLK, H2), lambda i: (0, i, 0)),
        pl.BlockSpec((BLK, H2), lambda i: (i, 0)),
        pl.BlockSpec((2, BLK), lambda i: (0, i)),
        pl.BlockSpec((1, H2), lambda i: (0, 0)),
        pl.BlockSpec((BLK, 1), lambda i: (i, 0)),
    ],
    out_specs=[
        pl.BlockSpec((G, H2), lambda i: (0, 0)),
        pl.BlockSpec((G, H2), lambda i: (0, 0)),
    ],
    out_shape=[
        jax.ShapeDtypeStruct((G, H2), jnp.float32),
        jax.ShapeDtypeStruct((G, H2), jnp.float32),
    ],
)

_OB = OUT // 10
_tc4 = pl.pallas_call(
    _tc4_body,
    grid=(10,),
    in_specs=[
        pl.BlockSpec((G, H2), lambda i: (0, 0)),
        pl.BlockSpec((G, H2), lambda i: (0, 0)),
        pl.BlockSpec((H2, FC1), lambda i: (0, 0)),
        pl.BlockSpec((1, FC1), lambda i: (0, 0)),
        pl.BlockSpec((FC1, _OB), lambda i: (0, i)),
        pl.BlockSpec((1, _OB), lambda i: (0, i)),
    ],
    out_specs=pl.BlockSpec((G, _OB), lambda i: (0, i)),
    out_shape=jax.ShapeDtypeStruct((G, OUT), jnp.float32),
)


def kernel(x, edge_index, batch, W1, b1, W2, b2, fcW1, fcb1, fcW2, fcb2):
    x_p = jnp.pad(x, ((0, NPAD - N), (0, 0)))
    bat_p = jnp.pad(batch, (0, NPAD - N), constant_values=G).reshape(NPAD, 1)
    zd = jnp.zeros((NPAD,), jnp.float32)
    z1 = jnp.zeros((NPAD, H1), jnp.float32)
    z2 = jnp.zeros((NPAD, H2), jnp.float32)

    pad_idx = N + (jnp.arange(E_PAD - E, dtype=jnp.int32) % (NPAD - N))
    ei2 = jnp.concatenate([edge_index,
                           jnp.stack([pad_idx, pad_idx])], axis=1)
    dst = ei2[1]
    _deg, _scat1, _scat2 = _sc_kernels()
    dp = _deg(dst, zd)
    hs1 = _tc1(x_p, W1, dp)
    p1 = _scat1(hs1, ei2, z1)
    hs2 = _tc2(p1, hs1, dp, b1.reshape(1, H1), W2)
    p2 = _scat2(hs2, ei2, z2)
    psum, cnt = _tc3(p2, hs2, dp, b2.reshape(1, H2), bat_p)
    out = _tc4(psum, cnt, fcW1, fcb1.reshape(1, FC1), fcW2, fcb2.reshape(1, OUT))
    return out


# speedup vs baseline: 28.3937x; 1.0044x over previous
import functools

import jax
import jax.numpy as jnp
from jax import lax
from jax.experimental import pallas as pl
from jax.experimental.pallas import tpu as pltpu
from jax.experimental.pallas import tpu_sc as plsc

N = 10000
E = 320000
IN = 128
H1 = 64
H2 = 128
FC1 = 1024
OUT = 6400
G = 64

NC, NS = 2, 16
NW = NC * NS
NPAD = 10240
RPT = NPAD // NS
EWP = 10240
E_PAD = NW * EWP
CH = 128
NCHUNK = EWP // CH
_MID = (NCHUNK - 6) // 6
_TAIL = 6 + 6 * _MID

BLK = 512
NB = NPAD // BLK

def _deg_body(dst_hbm, zd_hbm, dp_hbm, di0, di1, di2, ones_v, acc,
              ssem0, ssem1, isem0, isem1, isem2):
    c = lax.axis_index("c")
    s = lax.axis_index("s")
    wid = s * NC + c
    pltpu.sync_copy(zd_hbm.at[pl.ds(s * RPT, RPT)], acc.at[pl.ds(s * RPT, RPT)])
    for i in range(CH // 16):
        ones_v[pl.ds(i * 16, 16)] = jnp.full((16,), 1.0, jnp.float32)
    plsc.subcore_barrier()
    base = wid * EWP
    dib = (di0, di1, di2)
    ssems = (ssem0, ssem1)
    isems = (isem0, isem1, isem2)

    def idx_issue(t, k3):
        pltpu.async_copy(dst_hbm.at[pl.ds(base + t * CH, CH)], dib[k3], isems[k3])

    def idx_wait(t, k3):
        pltpu.make_async_copy(dst_hbm.at[pl.ds(base + t * CH, CH)],
                              dib[k3], isems[k3]).wait()

    def substep(t, k2, k3, first=False, idx_next=True):
        idx_wait(t, k3)
        pltpu.async_copy(ones_v, acc.at[dib[k3]], ssems[k2], add=True)
        if not first:
            pltpu.make_async_copy(ones_v, acc.at[dib[(k3 + 2) % 3]],
                                  ssems[1 - k2]).wait()
        if idx_next:
            idx_issue(t + 2, (k3 + 2) % 3)

    idx_issue(0, 0)
    idx_issue(1, 1)
    substep(0, 0, 0, first=True)
    for t in range(1, 6):
        substep(t, t % 2, t % 3)

    def step(jj, carry):
        t = 6 * jj
        for k in range(6):
            substep(t + k, k % 2, k % 3)
        return carry

    lax.fori_loop(1, _MID + 1, step, 0)
    for t in range(_TAIL, NCHUNK):
        substep(t, t % 2, t % 3, idx_next=(t + 2 < NCHUNK))
    lb = (NCHUNK - 1) % 2
    pltpu.make_async_copy(ones_v, acc.at[dib[(NCHUNK - 1) % 3]],
                          ssems[lb]).wait()
    plsc.subcore_barrier()
    pltpu.sync_copy(acc.at[pl.ds(s * RPT, RPT)], dp_hbm.at[c, pl.ds(s * RPT, RPT)])


def _scat_body(hs_hbm, ei_hbm, zr_hbm, part_hbm,
               ib0, ib1, ib2, rows0, rows1, acc,
               gsem, ssem0, ssem1, isem0, isem1, isem2):
    c = lax.axis_index("c")
    s = lax.axis_index("s")
    wid = s * NC + c
    pltpu.sync_copy(zr_hbm.at[pl.ds(s * RPT, RPT)], acc.at[pl.ds(s * RPT, RPT)])
    plsc.subcore_barrier()
    base = wid * EWP
    ib = (ib0, ib1, ib2)
    rbufs = (rows0, rows1)
    ssems = (ssem0, ssem1)
    isems = (isem0, isem1, isem2)

    def idx_issue(t, k3):
        pltpu.async_copy(ei_hbm.at[:, pl.ds(base + t * CH, CH)], ib[k3], isems[k3])

    def idx_wait(t, k3):
        pltpu.make_async_copy(ei_hbm.at[:, pl.ds(base + t * CH, CH)],
                              ib[k3], isems[k3]).wait()

    def substep(t, k2, k3, first=False, gather_next=True, idx_next=True):
        rc, ro = rbufs[k2], rbufs[1 - k2]
        pltpu.make_async_copy(hs_hbm.at[ib[k3].at[0]], rc, gsem).wait()
        pltpu.async_copy(rc, acc.at[ib[k3].at[1]], ssems[k2], add=True)
        if not first:
            pltpu.make_async_copy(ro, acc.at[ib[(k3 + 2) % 3].at[1]],
                                  ssems[1 - k2]).wait()
        if gather_next:
            idx_wait(t + 1, (k3 + 1) % 3)
            pltpu.async_copy(hs_hbm.at[ib[(k3 + 1) % 3].at[0]], ro, gsem)
        if idx_next:
            idx_issue(t + 2, (k3 + 2) % 3)

    idx_issue(0, 0)
    idx_issue(1, 1)
    idx_wait(0, 0)
    pltpu.async_copy(hs_hbm.at[ib0.at[0]], rows0, gsem)
    substep(0, 0, 0, first=True)
    for t in range(1, 6):
        substep(t, t % 2, t % 3)

    def step(jj, carry):
        t = 6 * jj
        for k in range(6):
            substep(t + k, k % 2, k % 3)
        return carry

    lax.fori_loop(1, _MID + 1, step, 0)
    for t in range(_TAIL, NCHUNK):
        substep(t, t % 2, t % 3,
                gather_next=(t + 1 < NCHUNK), idx_next=(t + 2 < NCHUNK))
    lb = (NCHUNK - 1) % 2
    pltpu.make_async_copy(rbufs[lb], acc.at[ib[(NCHUNK - 1) % 3].at[1]],
                          ssems[lb]).wait()
    plsc.subcore_barrier()
    pltpu.sync_copy(acc.at[pl.ds(s * RPT, RPT)],
                    part_hbm.at[c, pl.ds(s * RPT, RPT)])


@functools.lru_cache(maxsize=None)
def _sc_kernels():
    mesh = plsc.VectorSubcoreMesh(core_axis_name="c", subcore_axis_name="s",
                                  num_cores=NC, num_subcores=NS)
    deg = pl.kernel(
        _deg_body,
        out_type=jax.ShapeDtypeStruct((NC, NPAD), jnp.float32),
        mesh=mesh,
        scratch_types=[
            pltpu.VMEM((CH,), jnp.int32),
            pltpu.VMEM((CH,), jnp.int32),
            pltpu.VMEM((CH,), jnp.int32),
            pltpu.VMEM((CH,), jnp.float32),
            pltpu.VMEM_SHARED((NPAD,), jnp.float32),
            pltpu.SemaphoreType.DMA,
            pltpu.SemaphoreType.DMA,
            pltpu.SemaphoreType.DMA,
            pltpu.SemaphoreType.DMA,
            pltpu.SemaphoreType.DMA,
        ],
    )

    def make_scat(H, untiled):
        params = pltpu.CompilerParams(use_tc_tiling_on_sc=False) if untiled else None
        return pl.kernel(
            _scat_body,
            out_type=jax.ShapeDtypeStruct((NC, NPAD, H), jnp.float32),
            mesh=mesh,
            compiler_params=params,
            scratch_types=[
            pltpu.VMEM((2, CH), jnp.int32),
            pltpu.VMEM((2, CH), jnp.int32),
            pltpu.VMEM((2, CH), jnp.int32),
            pltpu.VMEM((CH, H), jnp.float32),
            pltpu.VMEM((CH, H), jnp.float32),
            pltpu.VMEM_SHARED((NPAD, H), jnp.float32),
            pltpu.SemaphoreType.DMA,
            pltpu.SemaphoreType.DMA,
            pltpu.SemaphoreType.DMA,
            pltpu.SemaphoreType.DMA,
            pltpu.SemaphoreType.DMA,
            pltpu.SemaphoreType.DMA,
        ],
        )

    return deg, make_scat(H1, True), make_scat(H2, False)


def _dinv_of(dp_ref):
    deg = dp_ref[0, :] + dp_ref[1, :] + 1.0
    return lax.rsqrt(deg)[:, None]


def _tc1_body(x_ref, w1_ref, dp_ref, hs1_ref):
    dinv = _dinv_of(dp_ref)
    hs1_ref[...] = dinv * jnp.dot(x_ref[...], w1_ref[...],
                                  preferred_element_type=jnp.float32)


def _tc2_body(p1_ref, hs1_ref, dp_ref, b1_ref, w2_ref, hs2_ref):
    dinv = _dinv_of(dp_ref)
    agg = p1_ref[0] + p1_ref[1] + hs1_ref[...]
    h = jnp.maximum(dinv * agg + b1_ref[...], 0.0)
    hs2 = dinv * jnp.dot(h, w2_ref[...], preferred_element_type=jnp.float32)
    row = pl.program_id(0) * BLK + lax.broadcasted_iota(jnp.int32, (BLK, 1), 0)
    hs2_ref[...] = jnp.where(row < N, hs2, 0.0)


def _tc34_body(p2_ref, hs2_ref, dp_ref, b2_ref, bat_ref,
               fw1_ref, fb1_ref, fw2_ref, fb2_ref, out_ref,
               psum_ref, cnt_ref, z_ref):
    i = pl.program_id(0)

    @pl.when(i == 0)
    def _():
        psum_ref[...] = jnp.zeros_like(psum_ref)
        cnt_ref[...] = jnp.zeros_like(cnt_ref)

    @pl.when(i < NB)
    def _():
        dinv = _dinv_of(dp_ref)
        h2 = jnp.maximum(
            dinv * (p2_ref[0] + p2_ref[1] + hs2_ref[...]) + b2_ref[...], 0.0)
        gid = lax.broadcasted_iota(jnp.int32, (BLK, G), 1)
        oh = (bat_ref[...] == gid).astype(jnp.float32)
        psum_ref[...] += lax.dot_general(oh, h2, (((0,), (0,)), ((), ())),
                                         preferred_element_type=jnp.float32)
        cnt_ref[...] += lax.dot_general(oh, jnp.ones((BLK, H2), jnp.float32),
                                        (((0,), (0,)), ((), ())),
                                        preferred_element_type=jnp.float32)

    @pl.when(i == NB)
    def _():
        pooled = psum_ref[...] / jnp.maximum(cnt_ref[...], 1.0)
        z_ref[...] = jnp.maximum(
            jnp.dot(pooled, fw1_ref[...],
                    preferred_element_type=jnp.float32) + fb1_ref[...], 0.0)

    @pl.when(i >= NB)
    def _():
        out_ref[...] = jnp.dot(z_ref[...], fw2_ref[...],
                               preferred_element_type=jnp.float32) + fb2_ref[...]


_tc1 = pl.pallas_call(
    _tc1_body,
    grid=(NB,),
    in_specs=[
        pl.BlockSpec((BLK, IN), lambda i: (i, 0)),
        pl.BlockSpec((IN, H1), lambda i: (0, 0)),
        pl.BlockSpec((2, BLK), lambda i: (0, i)),
    ],
    out_specs=pl.BlockSpec((BLK, H1), lambda i: (i, 0)),
    out_shape=jax.ShapeDtypeStruct((NPAD, H1), jnp.float32),
)

_tc2 = pl.pallas_call(
    _tc2_body,
    grid=(NB,),
    in_specs=[
        pl.BlockSpec((2, BLK, H1), lambda i: (0, i, 0)),
        pl.BlockSpec((BLK, H1), lambda i: (i, 0)),
        pl.BlockSpec((2, BLK), lambda i: (0, i)),
        pl.BlockSpec((1, H1), lambda i: (0, 0)),
        pl.BlockSpec((H1, H2), lambda i: (0, 0)),
    ],
    out_specs=pl.BlockSpec((BLK, H2), lambda i: (i, 0)),
    out_shape=jax.ShapeDtypeStruct((NPAD, H2), jnp.float32),
)

_OB = OUT // 10
_NB2 = NB + OUT // _OB


def _pool_i(i):
    return jnp.minimum(i, NB - 1)


def _head_i(i):
    return jnp.maximum(i - NB, 0)


_tc34 = pl.pallas_call(
    _tc34_body,
    grid=(_NB2,),
    in_specs=[
        pl.BlockSpec((2, BLK, H2), lambda i: (0, _pool_i(i), 0)),
        pl.BlockSpec((BLK, H2), lambda i: (_pool_i(i), 0)),
        pl.BlockSpec((2, BLK), lambda i: (0, _pool_i(i))),
        pl.BlockSpec((1, H2), lambda i: (0, 0)),
        pl.BlockSpec((BLK, 1), lambda i: (_pool_i(i), 0)),
        pl.BlockSpec((H2, FC1), lambda i: (0, 0)),
        pl.BlockSpec((1, FC1), lambda i: (0, 0)),
        pl.BlockSpec((FC1, _OB), lambda i: (0, _head_i(i))),
        pl.BlockSpec((1, _OB), lambda i: (0, _head_i(i))),
    ],
    out_specs=pl.BlockSpec((G, _OB), lambda i: (0, _head_i(i))),
    out_shape=jax.ShapeDtypeStruct((G, OUT), jnp.float32),
    scratch_shapes=[
        pltpu.VMEM((G, H2), jnp.float32),
        pltpu.VMEM((G, H2), jnp.float32),
        pltpu.VMEM((G, FC1), jnp.float32),
    ],
)


def kernel(x, edge_index, batch, W1, b1, W2, b2, fcW1, fcb1, fcW2, fcb2):
    x_p = jnp.pad(x, ((0, NPAD - N), (0, 0)))
    bat_p = jnp.pad(batch, (0, NPAD - N), constant_values=G).reshape(NPAD, 1)
    zd = jnp.zeros((NPAD,), jnp.float32)
    z1 = jnp.zeros((NPAD, H1), jnp.float32)
    z2 = jnp.zeros((NPAD, H2), jnp.float32)

    pad_idx = N + (jnp.arange(E_PAD - E, dtype=jnp.int32) % (NPAD - N))
    ei2 = jnp.concatenate([edge_index,
                           jnp.stack([pad_idx, pad_idx])], axis=1)
    dst = ei2[1]
    _deg, _scat1, _scat2 = _sc_kernels()
    dp = _deg(dst, zd)
    hs1 = _tc1(x_p, W1, dp)
    p1 = _scat1(hs1, ei2, z1)
    hs2 = _tc2(p1, hs1, dp, b1.reshape(1, H1), W2)
    p2 = _scat2(hs2, ei2, z2)
    out = _tc34(p2, hs2, dp, b2.reshape(1, H2), bat_p,
                fcW1, fcb1.reshape(1, FC1), fcW2, fcb2.reshape(1, OUT))
    return out


# speedup vs baseline: 28.5371x; 1.0051x over previous
import functools

import jax
import jax.numpy as jnp
from jax import lax
from jax.experimental import pallas as pl
from jax.experimental.pallas import tpu as pltpu
from jax.experimental.pallas import tpu_sc as plsc

N = 10000
E = 320000
IN = 128
H1 = 64
H2 = 128
FC1 = 1024
OUT = 6400
G = 64

NC, NS = 2, 16
NW = NC * NS
NPAD = 10240
RPT = NPAD // NS
EWP = 10240
E_PAD = NW * EWP
CH = 128
NCHUNK = EWP // CH
_MID = (NCHUNK - 6) // 6
_TAIL = 6 + 6 * _MID

BLK = 512
NB = NPAD // BLK

def _deg_body(dst_hbm, zd_hbm, dp_hbm, di0, di1, di2, ones_v, acc,
              ssem0, ssem1, isem0, isem1, isem2):
    c = lax.axis_index("c")
    s = lax.axis_index("s")
    wid = s * NC + c
    base = wid * EWP
    dib = (di0, di1, di2)
    ssems = (ssem0, ssem1)
    isems = (isem0, isem1, isem2)

    def idx_issue(t, k3):
        pltpu.async_copy(dst_hbm.at[pl.ds(base + t * CH, CH)], dib[k3], isems[k3])

    def idx_wait(t, k3):
        pltpu.make_async_copy(dst_hbm.at[pl.ds(base + t * CH, CH)],
                              dib[k3], isems[k3]).wait()

    def substep(t, k2, k3, first=False, idx_next=True):
        idx_wait(t, k3)
        pltpu.async_copy(ones_v, acc.at[dib[k3]], ssems[k2], add=True)
        if not first:
            pltpu.make_async_copy(ones_v, acc.at[dib[(k3 + 2) % 3]],
                                  ssems[1 - k2]).wait()
        if idx_next:
            idx_issue(t + 2, (k3 + 2) % 3)

    idx_issue(0, 0)
    idx_issue(1, 1)
    pltpu.sync_copy(zd_hbm.at[pl.ds(s * RPT, RPT)], acc.at[pl.ds(s * RPT, RPT)])
    for i in range(CH // 16):
        ones_v[pl.ds(i * 16, 16)] = jnp.full((16,), 1.0, jnp.float32)
    plsc.subcore_barrier()
    substep(0, 0, 0, first=True)
    for t in range(1, 6):
        substep(t, t % 2, t % 3)

    def step(jj, carry):
        t = 6 * jj
        for k in range(6):
            substep(t + k, k % 2, k % 3)
        return carry

    lax.fori_loop(1, _MID + 1, step, 0)
    for t in range(_TAIL, NCHUNK):
        substep(t, t % 2, t % 3, idx_next=(t + 2 < NCHUNK))
    lb = (NCHUNK - 1) % 2
    pltpu.make_async_copy(ones_v, acc.at[dib[(NCHUNK - 1) % 3]],
                          ssems[lb]).wait()
    plsc.subcore_barrier()
    pltpu.sync_copy(acc.at[pl.ds(s * RPT, RPT)], dp_hbm.at[c, pl.ds(s * RPT, RPT)])


def _scat_body(hs_hbm, ei_hbm, zr_hbm, part_hbm,
               ib0, ib1, ib2, rows0, rows1, acc,
               gsem, ssem0, ssem1, isem0, isem1, isem2):
    c = lax.axis_index("c")
    s = lax.axis_index("s")
    wid = s * NC + c
    base = wid * EWP
    ib = (ib0, ib1, ib2)
    rbufs = (rows0, rows1)
    ssems = (ssem0, ssem1)
    isems = (isem0, isem1, isem2)

    def idx_issue(t, k3):
        pltpu.async_copy(ei_hbm.at[:, pl.ds(base + t * CH, CH)], ib[k3], isems[k3])

    def idx_wait(t, k3):
        pltpu.make_async_copy(ei_hbm.at[:, pl.ds(base + t * CH, CH)],
                              ib[k3], isems[k3]).wait()

    def substep(t, k2, k3, first=False, gather_next=True, idx_next=True):
        rc, ro = rbufs[k2], rbufs[1 - k2]
        pltpu.make_async_copy(hs_hbm.at[ib[k3].at[0]], rc, gsem).wait()
        pltpu.async_copy(rc, acc.at[ib[k3].at[1]], ssems[k2], add=True)
        if not first:
            pltpu.make_async_copy(ro, acc.at[ib[(k3 + 2) % 3].at[1]],
                                  ssems[1 - k2]).wait()
        if gather_next:
            idx_wait(t + 1, (k3 + 1) % 3)
            pltpu.async_copy(hs_hbm.at[ib[(k3 + 1) % 3].at[0]], ro, gsem)
        if idx_next:
            idx_issue(t + 2, (k3 + 2) % 3)

    idx_issue(0, 0)
    idx_issue(1, 1)
    idx_wait(0, 0)
    pltpu.async_copy(hs_hbm.at[ib0.at[0]], rows0, gsem)
    pltpu.sync_copy(zr_hbm.at[pl.ds(s * RPT, RPT)], acc.at[pl.ds(s * RPT, RPT)])
    plsc.subcore_barrier()
    substep(0, 0, 0, first=True)
    for t in range(1, 6):
        substep(t, t % 2, t % 3)

    def step(jj, carry):
        t = 6 * jj
        for k in range(6):
            substep(t + k, k % 2, k % 3)
        return carry

    lax.fori_loop(1, _MID + 1, step, 0)
    for t in range(_TAIL, NCHUNK):
        substep(t, t % 2, t % 3,
                gather_next=(t + 1 < NCHUNK), idx_next=(t + 2 < NCHUNK))
    lb = (NCHUNK - 1) % 2
    pltpu.make_async_copy(rbufs[lb], acc.at[ib[(NCHUNK - 1) % 3].at[1]],
                          ssems[lb]).wait()
    plsc.subcore_barrier()
    pltpu.sync_copy(acc.at[pl.ds(s * RPT, RPT)],
                    part_hbm.at[c, pl.ds(s * RPT, RPT)])


@functools.lru_cache(maxsize=None)
def _sc_kernels():
    mesh = plsc.VectorSubcoreMesh(core_axis_name="c", subcore_axis_name="s",
                                  num_cores=NC, num_subcores=NS)
    deg = pl.kernel(
        _deg_body,
        out_type=jax.ShapeDtypeStruct((NC, NPAD), jnp.float32),
        mesh=mesh,
        scratch_types=[
            pltpu.VMEM((CH,), jnp.int32),
            pltpu.VMEM((CH,), jnp.int32),
            pltpu.VMEM((CH,), jnp.int32),
            pltpu.VMEM((CH,), jnp.float32),
            pltpu.VMEM_SHARED((NPAD,), jnp.float32),
            pltpu.SemaphoreType.DMA,
            pltpu.SemaphoreType.DMA,
            pltpu.SemaphoreType.DMA,
            pltpu.SemaphoreType.DMA,
            pltpu.SemaphoreType.DMA,
        ],
    )

    def make_scat(H, untiled):
        params = pltpu.CompilerParams(use_tc_tiling_on_sc=False) if untiled else None
        return pl.kernel(
            _scat_body,
            out_type=jax.ShapeDtypeStruct((NC, NPAD, H), jnp.float32),
            mesh=mesh,
            compiler_params=params,
            scratch_types=[
            pltpu.VMEM((2, CH), jnp.int32),
            pltpu.VMEM((2, CH), jnp.int32),
            pltpu.VMEM((2, CH), jnp.int32),
            pltpu.VMEM((CH, H), jnp.float32),
            pltpu.VMEM((CH, H), jnp.float32),
            pltpu.VMEM_SHARED((NPAD, H), jnp.float32),
            pltpu.SemaphoreType.DMA,
            pltpu.SemaphoreType.DMA,
            pltpu.SemaphoreType.DMA,
            pltpu.SemaphoreType.DMA,
            pltpu.SemaphoreType.DMA,
            pltpu.SemaphoreType.DMA,
        ],
        )

    return deg, make_scat(H1, True), make_scat(H2, False)


def _dinv_of(dp_ref):
    deg = dp_ref[0, :] + dp_ref[1, :] + 1.0
    return lax.rsqrt(deg)[:, None]


def _tc1_body(x_ref, w1_ref, dp_ref, hs1_ref):
    dinv = _dinv_of(dp_ref)
    hs1_ref[...] = dinv * jnp.dot(x_ref[...], w1_ref[...],
                                  preferred_element_type=jnp.float32)


def _tc2_body(p1_ref, hs1_ref, dp_ref, b1_ref, w2_ref, hs2_ref):
    dinv = _dinv_of(dp_ref)
    agg = p1_ref[0] + p1_ref[1] + hs1_ref[...]
    h = jnp.maximum(dinv * agg + b1_ref[...], 0.0)
    hs2 = dinv * jnp.dot(h, w2_ref[...], preferred_element_type=jnp.float32)
    row = pl.program_id(0) * BLK + lax.broadcasted_iota(jnp.int32, (BLK, 1), 0)
    hs2_ref[...] = jnp.where(row < N, hs2, 0.0)


def _tc34_body(p2_ref, hs2_ref, dp_ref, b2_ref, bat_ref,
               fw1_ref, fb1_ref, fw2_ref, fb2_ref, out_ref,
               psum_ref, cnt_ref, z_ref):
    i = pl.program_id(0)

    @pl.when(i == 0)
    def _():
        psum_ref[...] = jnp.zeros_like(psum_ref)
        cnt_ref[...] = jnp.zeros_like(cnt_ref)

    @pl.when(i < NB)
    def _():
        dinv = _dinv_of(dp_ref)
        h2 = jnp.maximum(
            dinv * (p2_ref[0] + p2_ref[1] + hs2_ref[...]) + b2_ref[...], 0.0)
        gid = lax.broadcasted_iota(jnp.int32, (BLK, G), 1)
        oh = (bat_ref[...] == gid).astype(jnp.float32)
        psum_ref[...] += lax.dot_general(oh, h2, (((0,), (0,)), ((), ())),
                                         preferred_element_type=jnp.float32)
        cnt_ref[...] += lax.dot_general(oh, jnp.ones((BLK, H2), jnp.float32),
                                        (((0,), (0,)), ((), ())),
                                        preferred_element_type=jnp.float32)

    @pl.when(i == NB)
    def _():
        pooled = psum_ref[...] / jnp.maximum(cnt_ref[...], 1.0)
        z_ref[...] = jnp.maximum(
            jnp.dot(pooled, fw1_ref[...],
                    preferred_element_type=jnp.float32) + fb1_ref[...], 0.0)

    @pl.when(i >= NB)
    def _():
        out_ref[...] = jnp.dot(z_ref[...], fw2_ref[...],
                               preferred_element_type=jnp.float32) + fb2_ref[...]


_tc1 = pl.pallas_call(
    _tc1_body,
    grid=(NB,),
    in_specs=[
        pl.BlockSpec((BLK, IN), lambda i: (i, 0)),
        pl.BlockSpec((IN, H1), lambda i: (0, 0)),
        pl.BlockSpec((2, BLK), lambda i: (0, i)),
    ],
    out_specs=pl.BlockSpec((BLK, H1), lambda i: (i, 0)),
    out_shape=jax.ShapeDtypeStruct((NPAD, H1), jnp.float32),
)

_tc2 = pl.pallas_call(
    _tc2_body,
    grid=(NB,),
    in_specs=[
        pl.BlockSpec((2, BLK, H1), lambda i: (0, i, 0)),
        pl.BlockSpec((BLK, H1), lambda i: (i, 0)),
        pl.BlockSpec((2, BLK), lambda i: (0, i)),
        pl.BlockSpec((1, H1), lambda i: (0, 0)),
        pl.BlockSpec((H1, H2), lambda i: (0, 0)),
    ],
    out_specs=pl.BlockSpec((BLK, H2), lambda i: (i, 0)),
    out_shape=jax.ShapeDtypeStruct((NPAD, H2), jnp.float32),
)

_OB = OUT // 10
_NB2 = NB + OUT // _OB


def _pool_i(i):
    return jnp.minimum(i, NB - 1)


def _head_i(i):
    return jnp.maximum(i - NB, 0)


_tc34 = pl.pallas_call(
    _tc34_body,
    grid=(_NB2,),
    in_specs=[
        pl.BlockSpec((2, BLK, H2), lambda i: (0, _pool_i(i), 0)),
        pl.BlockSpec((BLK, H2), lambda i: (_pool_i(i), 0)),
        pl.BlockSpec((2, BLK), lambda i: (0, _pool_i(i))),
        pl.BlockSpec((1, H2), lambda i: (0, 0)),
        pl.BlockSpec((BLK, 1), lambda i: (_pool_i(i), 0)),
        pl.BlockSpec((H2, FC1), lambda i: (0, 0)),
        pl.BlockSpec((1, FC1), lambda i: (0, 0)),
        pl.BlockSpec((FC1, _OB), lambda i: (0, _head_i(i))),
        pl.BlockSpec((1, _OB), lambda i: (0, _head_i(i))),
    ],
    out_specs=pl.BlockSpec((G, _OB), lambda i: (0, _head_i(i))),
    out_shape=jax.ShapeDtypeStruct((G, OUT), jnp.float32),
    scratch_shapes=[
        pltpu.VMEM((G, H2), jnp.float32),
        pltpu.VMEM((G, H2), jnp.float32),
        pltpu.VMEM((G, FC1), jnp.float32),
    ],
)


def kernel(x, edge_index, batch, W1, b1, W2, b2, fcW1, fcb1, fcW2, fcb2):
    x_p = jnp.pad(x, ((0, NPAD - N), (0, 0)))
    bat_p = jnp.pad(batch, (0, NPAD - N), constant_values=G).reshape(NPAD, 1)
    zd = jnp.zeros((NPAD,), jnp.float32)
    z1 = jnp.zeros((NPAD, H1), jnp.float32)
    z2 = jnp.zeros((NPAD, H2), jnp.float32)

    pad_idx = N + (jnp.arange(E_PAD - E, dtype=jnp.int32) % (NPAD - N))
    ei2 = jnp.concatenate([edge_index,
                           jnp.stack([pad_idx, pad_idx])], axis=1)
    dst = ei2[1]
    _deg, _scat1, _scat2 = _sc_kernels()
    dp = _deg(dst, zd)
    hs1 = _tc1(x_p, W1, dp)
    p1 = _scat1(hs1, ei2, z1)
    hs2 = _tc2(p1, hs1, dp, b1.reshape(1, H1), W2)
    p2 = _scat2(hs2, ei2, z2)
    out = _tc34(p2, hs2, dp, b2.reshape(1, H2), bat_p,
                fcW1, fcb1.reshape(1, FC1), fcW2, fcb2.reshape(1, OUT))
    return out
